# Initial kernel scaffold; baseline (speedup 1.0000x reference)
import functools

import jax
import jax.numpy as jnp
from jax import lax
from jax.experimental import pallas as pl
from jax.experimental.pallas import tpu as pltpu
from jax.experimental.pallas import tpu_sc as plsc

N = 50000
E = 1600000
G = 64
F = 32
NC, NS = 2, 16
NW = NC * NS
EPW = E // NW
CH = 1000
NCH = EPW // CH
ROWS_W = N // NS
DEGW = 16

BLK = 1000
NB = N // BLK


def _deg_body(dst_hbm, ones_hbm, zeros_hbm, out_hbm, didx, ones_v, deg_sh):
    c = lax.axis_index("c")
    s = lax.axis_index("s")
    wid = s * NC + c
    pltpu.sync_copy(zeros_hbm.at[pl.ds(s * ROWS_W, ROWS_W)],
                    deg_sh.at[pl.ds(s * ROWS_W, ROWS_W)])
    pltpu.sync_copy(ones_hbm, ones_v)
    plsc.subcore_barrier()

    def body(i, carry):
        base = wid * EPW + i * CH
        pltpu.sync_copy(dst_hbm.at[pl.ds(base, CH)], didx)
        pltpu.sync_copy(ones_v, deg_sh.at[didx], add=True)
        return carry

    lax.fori_loop(0, NCH, body, 0)
    plsc.subcore_barrier()
    pltpu.sync_copy(deg_sh.at[pl.ds(s * ROWS_W, ROWS_W)],
                    out_hbm.at[c, pl.ds(s * ROWS_W, ROWS_W)])


_deg_call = pl.kernel(
    _deg_body,
    out_type=jax.ShapeDtypeStruct((NC, N, DEGW), jnp.float32),
    mesh=plsc.VectorSubcoreMesh(core_axis_name="c", subcore_axis_name="s"),
    scratch_types=[
        pltpu.VMEM((CH,), jnp.int32),
        pltpu.VMEM((CH, DEGW), jnp.float32),
        pltpu.VMEM_SHARED((N, DEGW), jnp.float32),
    ],
)


def _agg_body(hs_hbm, src_hbm, dst_hbm, zeros_hbm, out_hbm,
              sidx, didx, rows, agg_sh, sem):
    c = lax.axis_index("c")
    s = lax.axis_index("s")
    wid = s * NC + c
    pltpu.sync_copy(zeros_hbm.at[pl.ds(s * ROWS_W, ROWS_W)],
                    agg_sh.at[pl.ds(s * ROWS_W, ROWS_W)])
    plsc.subcore_barrier()

    def body(i, carry):
        base = wid * EPW + i * CH
        pltpu.sync_copy(src_hbm.at[pl.ds(base, CH)], sidx)
        pltpu.sync_copy(dst_hbm.at[pl.ds(base, CH)], didx)
        pltpu.async_copy(hs_hbm.at[sidx], rows, sem).wait()
        pltpu.sync_copy(rows, agg_sh.at[didx], add=True)
        return carry

    lax.fori_loop(0, NCH, body, 0)
    plsc.subcore_barrier()
    pltpu.sync_copy(agg_sh.at[pl.ds(s * ROWS_W, ROWS_W)],
                    out_hbm.at[c, pl.ds(s * ROWS_W, ROWS_W)])


_agg_call = pl.kernel(
    _agg_body,
    out_type=jax.ShapeDtypeStruct((NC, N, F), jnp.float32),
    mesh=plsc.VectorSubcoreMesh(core_axis_name="c", subcore_axis_name="s"),
    scratch_types=[
        pltpu.VMEM((CH,), jnp.int32),
        pltpu.VMEM((CH,), jnp.int32),
        pltpu.VMEM((CH, F), jnp.float32),
        pltpu.VMEM_SHARED((N, F), jnp.float32),
        pltpu.SemaphoreType.DMA,
    ],
)


def _k1_body(x_ref, degp_ref, g0_ref, b0_ref, w_ref,
             h_ref, hs_ref, dis_ref, acc_ref):
    ph = pl.program_id(0)
    j = pl.program_id(1)
    x = x_ref[...]
    dp = degp_ref[...]
    deg = 1.0 + dp[0, :, 0:1] + dp[1, :, 0:1]
    dis = lax.rsqrt(deg)
    dis_ref[...] = dis

    @pl.when(jnp.logical_and(ph == 0, j == 0))
    def _():
        acc_ref[...] = jnp.zeros_like(acc_ref)

    @pl.when(ph == 0)
    def _():
        acc_ref[0:1, 0:13] += jnp.sum(x, axis=0, keepdims=True)
        acc_ref[1:2, 0:13] += jnp.sum(x * x, axis=0, keepdims=True)

    m = acc_ref[0:1, 0:13] / N
    v = acc_ref[1:2, 0:13] / N - m * m
    xn = (x - m) * lax.rsqrt(v + 1e-5) * g0_ref[...] + b0_ref[...]
    h = jnp.dot(xn, w_ref[...], preferred_element_type=jnp.float32)
    h_ref[...] = h
    hs_ref[...] = h * dis


def _make_k1():
    return pl.pallas_call(
        _k1_body,
        grid=(2, NB),
        in_specs=[
            pl.BlockSpec((BLK, 13), lambda ph, j: (j, 0)),
            pl.BlockSpec((NC, BLK, DEGW), lambda ph, j: (0, j, 0)),
            pl.BlockSpec((1, 13), lambda ph, j: (0, 0)),
            pl.BlockSpec((1, 13), lambda ph, j: (0, 0)),
            pl.BlockSpec((13, F), lambda ph, j: (0, 0)),
        ],
        out_specs=[
            pl.BlockSpec((BLK, F), lambda ph, j: (j, 0)),
            pl.BlockSpec((BLK, F), lambda ph, j: (j, 0)),
            pl.BlockSpec((BLK, 1), lambda ph, j: (j, 0)),
        ],
        out_shape=[
            jax.ShapeDtypeStruct((N, F), jnp.float32),
            jax.ShapeDtypeStruct((N, F), jnp.float32),
            jax.ShapeDtypeStruct((N, 1), jnp.float32),
        ],
        scratch_shapes=[pltpu.VMEM((8, 128), jnp.float32)],
    )


def _layer_body(p_ref, h_ref, dis_ref, fb_ref, g_ref, b_ref, w_ref,
                hn_ref, hsn_ref, acc_ref):
    ph = pl.program_id(0)
    j = pl.program_id(1)
    p = p_ref[...]
    h = h_ref[...]
    dis = dis_ref[...]
    u = dis * (p[0] + p[1]) + h * (dis * dis) + fb_ref[...]
    u = jnp.maximum(u, 0.0)

    @pl.when(jnp.logical_and(ph == 0, j == 0))
    def _():
        acc_ref[...] = jnp.zeros_like(acc_ref)

    @pl.when(ph == 0)
    def _():
        acc_ref[0:1, 0:F] += jnp.sum(u, axis=0, keepdims=True)
        acc_ref[1:2, 0:F] += jnp.sum(u * u, axis=0, keepdims=True)

    m = acc_ref[0:1, 0:F] / N
    v = acc_ref[1:2, 0:F] / N - m * m
    mx = (u - m) * lax.rsqrt(v + 1e-5) * g_ref[...] + b_ref[...]
    hn = jnp.dot(mx, w_ref[...], preferred_element_type=jnp.float32)
    hn_ref[...] = hn
    hsn_ref[...] = hn * dis


def _make_layer():
    return pl.pallas_call(
        _layer_body,
        grid=(2, NB),
        in_specs=[
            pl.BlockSpec((NC, BLK, F), lambda ph, j: (0, j, 0)),
            pl.BlockSpec((BLK, F), lambda ph, j: (j, 0)),
            pl.BlockSpec((BLK, 1), lambda ph, j: (j, 0)),
            pl.BlockSpec((1, F), lambda ph, j: (0, 0)),
            pl.BlockSpec((1, F), lambda ph, j: (0, 0)),
            pl.BlockSpec((1, F), lambda ph, j: (0, 0)),
            pl.BlockSpec((F, F), lambda ph, j: (0, 0)),
        ],
        out_specs=[
            pl.BlockSpec((BLK, F), lambda ph, j: (j, 0)),
            pl.BlockSpec((BLK, F), lambda ph, j: (j, 0)),
        ],
        out_shape=[
            jax.ShapeDtypeStruct((N, F), jnp.float32),
            jax.ShapeDtypeStruct((N, F), jnp.float32),
        ],
        scratch_shapes=[pltpu.VMEM((8, 128), jnp.float32)],
    )


def _conn_body(adj_ref, cx_ref, w1r_ref, b1_ref, w1t_ref, a1_ref,
               w2r_ref, b2_ref, w2t_ref, a2_ref,
               w3r_ref, b3_ref, w3t_ref, a3_ref, out_ref):
    a = adj_ref[0]
    x = cx_ref[0]

    def prelu(z, al):
        return jnp.where(z >= 0, z, al * z)

    t = jnp.dot(a, x, preferred_element_type=jnp.float32)
    h = (jnp.dot(t, w1r_ref[...], preferred_element_type=jnp.float32)
         + b1_ref[...]
         + jnp.dot(x, w1t_ref[...], preferred_element_type=jnp.float32))
    h = prelu(h, a1_ref[...])
    t = jnp.dot(a, h, preferred_element_type=jnp.float32)
    h = (jnp.dot(t, w2r_ref[...], preferred_element_type=jnp.float32)
         + b2_ref[...]
         + jnp.dot(h, w2t_ref[...], preferred_element_type=jnp.float32))
    h = prelu(h, a2_ref[...])
    t = jnp.dot(a, h, preferred_element_type=jnp.float32)
    h = (jnp.dot(t, w3r_ref[...], preferred_element_type=jnp.float32)
         + b3_ref[...]
         + jnp.dot(h, w3t_ref[...], preferred_element_type=jnp.float32))
    h = prelu(h, a3_ref[...])
    out_ref[0] = h


def _make_conn():
    full = lambda r, c: pl.BlockSpec((r, c), lambda g: (0, 0))
    return pl.pallas_call(
        _conn_body,
        grid=(G,),
        in_specs=[
            pl.BlockSpec((1, 87, 87), lambda g: (g, 0, 0)),
            pl.BlockSpec((1, 87, 10), lambda g: (g, 0, 0)),
            full(10, 20), full(1, 20), full(10, 20), full(1, 1),
            full(20, 20), full(1, 20), full(20, 20), full(1, 1),
            full(20, 5), full(1, 5), full(20, 5), full(1, 1),
        ],
        out_specs=pl.BlockSpec((1, 87, 5), lambda g: (g, 0, 0)),
        out_shape=jax.ShapeDtypeStruct((G, 87, 5), jnp.float32),
    )


def _final_body(p_ref, h_ref, dis_ref, fb_ref, g_ref, b_ref, batch_ref,
                conn_ref, w1a_ref, w1b_ref, hb1_ref, hg_ref, hbb_ref,
                w2_ref, hb2_ref, out_ref, acc_ref, seg_ref, cnt_ref):
    j = pl.program_id(0)
    p = p_ref[...]
    h = h_ref[...]
    dis = dis_ref[...]
    u = dis * (p[0] + p[1]) + h * (dis * dis) + fb_ref[...]
    u = jnp.maximum(u, 0.0)

    @pl.when(j == 0)
    def _():
        acc_ref[...] = jnp.zeros_like(acc_ref)
        seg_ref[...] = jnp.zeros_like(seg_ref)
        cnt_ref[...] = jnp.zeros_like(cnt_ref)

    acc_ref[0:1, 0:F] += jnp.sum(u, axis=0, keepdims=True)
    acc_ref[1:2, 0:F] += jnp.sum(u * u, axis=0, keepdims=True)

    b = batch_ref[...]
    mask = (lax.broadcasted_iota(jnp.int32, (G, BLK), 0) == b
            ).astype(jnp.float32)
    seg_ref[...] += jnp.dot(mask, u, preferred_element_type=jnp.float32)
    cnt_ref[...] += jnp.sum(mask, axis=1, keepdims=True)

    m = acc_ref[0:1, 0:F] / N
    v = acc_ref[1:2, 0:F] / N - m * m
    rstd = lax.rsqrt(v + 1e-5)
    cnt = jnp.maximum(cnt_ref[...], 1.0)
    mean_u = seg_ref[...] / cnt
    mesh_feat = (mean_u - m) * rstd * g_ref[...] + b_ref[...]
    z = (jnp.dot(mesh_feat, w1a_ref[...], preferred_element_type=jnp.float32)
         + jnp.dot(conn_ref[...], w1b_ref[...],
                   preferred_element_type=jnp.float32)
         + hb1_ref[...])
    z = jnp.maximum(z, 0.0)
    zm = jnp.mean(z, axis=0, keepdims=True)
    zv = jnp.mean(z * z, axis=0, keepdims=True) - zm * zm
    zn = (z - zm) * lax.rsqrt(zv + 1e-5) * hg_ref[...] + hbb_ref[...]
    out_ref[...] = (jnp.dot(zn, w2_ref[...], preferred_element_type=jnp.float32)
                    + hb2_ref[...])


def _make_final():
    full = lambda r, c: pl.BlockSpec((r, c), lambda j: (0, 0))
    return pl.pallas_call(
        _final_body,
        grid=(NB,),
        in_specs=[
            pl.BlockSpec((NC, BLK, F), lambda j: (0, j, 0)),
            pl.BlockSpec((BLK, F), lambda j: (j, 0)),
            pl.BlockSpec((BLK, 1), lambda j: (j, 0)),
            full(1, F), full(1, F), full(1, F),
            pl.BlockSpec((1, BLK), lambda j: (0, j)),
            full(G, 435), full(F, 10), full(435, 10), full(1, 10),
            full(1, 10), full(1, 10), full(10, 1), full(1, 1),
        ],
        out_specs=pl.BlockSpec((G, 1), lambda j: (0, 0)),
        out_shape=jax.ShapeDtypeStruct((G, 1), jnp.float32),
        scratch_shapes=[
            pltpu.VMEM((8, 128), jnp.float32),
            pltpu.VMEM((G, F), jnp.float32),
            pltpu.VMEM((G, 1), jnp.float32),
        ],
    )


_k1 = _make_k1()
_klayer = _make_layer()
_kconn = _make_conn()
_kfinal = _make_final()


def kernel(mesh_pos, mesh_norm, mesh_dha, mesh_x, mesh_edge_index, mesh_batch,
           conn_x, conn_adj,
           cw1_rel, cb1, cw1_root, ca1,
           cw2_rel, cb2, cw2_root, ca2,
           cw3_rel, cb3, cw3_root, ca3,
           bn0_g, bn0_b,
           fw1, fb1, bn1_g, bn1_b,
           fw2, fb2, bn2_g, bn2_b,
           fw3, fb3, bn3_g, bn3_b,
           fw4, fb4, bn4_g, bn4_b,
           hw1, hb1, hbn_g, hbn_b, hw2, hb2):
    r = lambda a: a.reshape(1, -1)
    x13 = jnp.concatenate([mesh_pos, mesh_norm, mesh_dha, mesh_x], axis=1)
    src = mesh_edge_index[0]
    dst = mesh_edge_index[1]
    zeros_f = jnp.zeros((N, F), jnp.float32)
    zeros_d = jnp.zeros((N, DEGW), jnp.float32)
    ones_d = jnp.ones((CH, DEGW), jnp.float32)

    degp = _deg_call(dst, ones_d, zeros_d)

    conn3 = _kconn(conn_adj, conn_x,
                   cw1_rel, r(cb1), cw1_root, r(ca1),
                   cw2_rel, r(cb2), cw2_root, r(ca2),
                   cw3_rel, r(cb3), cw3_root, r(ca3))
    conn_feat = conn3.reshape(G, 435)

    h, hs, dis = _k1(x13, degp, r(bn0_g), r(bn0_b), fw1)

    fbs = (fb1, fb2, fb3)
    gs = (bn1_g, bn2_g, bn3_g)
    bs = (bn1_b, bn2_b, bn3_b)
    ws = (fw2, fw3, fw4)
    for i in range(3):
        p = _agg_call(hs, src, dst, zeros_f)
        h, hs = _klayer(p, h, dis, r(fbs[i]), r(gs[i]), r(bs[i]), ws[i])

    p = _agg_call(hs, src, dst, zeros_f)
    out = _kfinal(p, h, dis, r(fb4), r(bn4_g), r(bn4_b),
                  mesh_batch.reshape(1, N), conn_feat,
                  hw1[:F], hw1[F:], r(hb1), r(hbn_g), r(hbn_b), hw2, r(hb2))
    return out


# speedup vs baseline: 18.9432x; 18.9432x over previous
import functools

import jax
import jax.numpy as jnp
from jax import lax
from jax.experimental import pallas as pl
from jax.experimental.pallas import tpu as pltpu
from jax.experimental.pallas import tpu_sc as plsc

N = 50000
E = 1600000
G = 64
F = 32
NC, NS = 2, 16
NW = NC * NS
EPW = E // NW
CH = 1000
NCH = EPW // CH
NPAD = 50048
ROWS_W = NPAD // NS
DEGW = 16

BLK = 1000
NB = N // BLK


FH = F // NC
EPS = E // NS
NCHS = EPS // CH


def _agg_body(hs0_hbm, hs1_hbm, src_hbm, dst_hbm, zeros_hbm, out_hbm,
              sidx, didx, rows, agg_sh, sem):
    c = lax.axis_index("c")
    s = lax.axis_index("s")
    pltpu.sync_copy(zeros_hbm.at[pl.ds(s * ROWS_W, ROWS_W)],
                    agg_sh.at[pl.ds(s * ROWS_W, ROWS_W)])
    plsc.subcore_barrier()

    def body(i, carry):
        base = s * EPS + i * CH
        pltpu.sync_copy(src_hbm.at[pl.ds(base, CH)], sidx)
        pltpu.sync_copy(dst_hbm.at[pl.ds(base, CH)], didx)

        @pl.when(c == 0)
        def _():
            pltpu.async_copy(hs0_hbm.at[sidx], rows, sem).wait()

        @pl.when(c == 1)
        def _():
            pltpu.async_copy(hs1_hbm.at[sidx], rows, sem).wait()

        pltpu.sync_copy(rows, agg_sh.at[didx], add=True)
        return carry

    lax.fori_loop(0, NCHS, body, 0)
    plsc.subcore_barrier()
    pltpu.sync_copy(agg_sh.at[pl.ds(s * ROWS_W, ROWS_W)],
                    out_hbm.at[pl.ds(s * ROWS_W, ROWS_W), pl.ds(c * FH, FH)])


@functools.lru_cache(maxsize=None)
def _make_agg():
    return pl.kernel(
        _agg_body,
        out_type=jax.ShapeDtypeStruct((NPAD, F), jnp.float32),
        mesh=plsc.VectorSubcoreMesh(core_axis_name="c", subcore_axis_name="s"),
        scratch_types=[
            pltpu.VMEM((CH,), jnp.int32),
            pltpu.VMEM((CH,), jnp.int32),
            pltpu.VMEM((CH, FH), jnp.float32),
            pltpu.VMEM_SHARED((NPAD, FH), jnp.float32),
            pltpu.SemaphoreType.DMA,
        ],
        compiler_params=pltpu.CompilerParams(use_tc_tiling_on_sc=False),
    )


def _k1_body(x_ref, degp_ref, g0_ref, b0_ref, w_ref,
             h_ref, hs0_ref, hs1_ref, dis_ref, acc_ref):
    ph = pl.program_id(0)
    j = pl.program_id(1)
    x = x_ref[...]
    dp = degp_ref[...]
    deg = 1.0 + dp[:, 0:1]
    dis = lax.rsqrt(deg)
    dis_ref[...] = dis

    @pl.when(jnp.logical_and(ph == 0, j == 0))
    def _():
        acc_ref[...] = jnp.zeros_like(acc_ref)

    @pl.when(ph == 0)
    def _():
        acc_ref[0:1, 0:13] += jnp.sum(x, axis=0, keepdims=True)
        acc_ref[1:2, 0:13] += jnp.sum(x * x, axis=0, keepdims=True)

    m = acc_ref[0:1, 0:13] / N
    v = acc_ref[1:2, 0:13] / N - m * m
    xn = (x - m) * lax.rsqrt(v + 1e-5) * g0_ref[...] + b0_ref[...]
    h = jnp.dot(xn, w_ref[...], preferred_element_type=jnp.float32)
    h_ref[...] = h
    hsc = h * dis
    hs0_ref[...] = hsc[:, :FH]
    hs1_ref[...] = hsc[:, FH:]


def _make_k1():
    return pl.pallas_call(
        _k1_body,
        grid=(2, NB),
        in_specs=[
            pl.BlockSpec((BLK, 13), lambda ph, j: (j, 0)),
            pl.BlockSpec((BLK, F), lambda ph, j: (j, 0)),
            pl.BlockSpec((1, 13), lambda ph, j: (0, 0)),
            pl.BlockSpec((1, 13), lambda ph, j: (0, 0)),
            pl.BlockSpec((13, F), lambda ph, j: (0, 0)),
        ],
        out_specs=[
            pl.BlockSpec((BLK, F), lambda ph, j: (j, 0)),
            pl.BlockSpec((BLK, FH), lambda ph, j: (j, 0)),
            pl.BlockSpec((BLK, FH), lambda ph, j: (j, 0)),
            pl.BlockSpec((BLK, 1), lambda ph, j: (j, 0)),
        ],
        out_shape=[
            jax.ShapeDtypeStruct((N, F), jnp.float32),
            jax.ShapeDtypeStruct((N, FH), jnp.float32),
            jax.ShapeDtypeStruct((N, FH), jnp.float32),
            jax.ShapeDtypeStruct((N, 1), jnp.float32),
        ],
        scratch_shapes=[pltpu.VMEM((8, 128), jnp.float32)],
    )


def _layer_body(p_ref, h_ref, dis_ref, fb_ref, g_ref, b_ref, w_ref,
                hn_ref, hs0_ref, hs1_ref, acc_ref):
    ph = pl.program_id(0)
    j = pl.program_id(1)
    p = p_ref[...]
    h = h_ref[...]
    dis = dis_ref[...]
    u = dis * p + h * (dis * dis) + fb_ref[...]
    u = jnp.maximum(u, 0.0)

    @pl.when(jnp.logical_and(ph == 0, j == 0))
    def _():
        acc_ref[...] = jnp.zeros_like(acc_ref)

    @pl.when(ph == 0)
    def _():
        acc_ref[0:1, 0:F] += jnp.sum(u, axis=0, keepdims=True)
        acc_ref[1:2, 0:F] += jnp.sum(u * u, axis=0, keepdims=True)

    m = acc_ref[0:1, 0:F] / N
    v = acc_ref[1:2, 0:F] / N - m * m
    mx = (u - m) * lax.rsqrt(v + 1e-5) * g_ref[...] + b_ref[...]
    hn = jnp.dot(mx, w_ref[...], preferred_element_type=jnp.float32)
    hn_ref[...] = hn
    hsc = hn * dis
    hs0_ref[...] = hsc[:, :FH]
    hs1_ref[...] = hsc[:, FH:]


def _make_layer():
    return pl.pallas_call(
        _layer_body,
        grid=(2, NB),
        in_specs=[
            pl.BlockSpec((BLK, F), lambda ph, j: (j, 0)),
            pl.BlockSpec((BLK, F), lambda ph, j: (j, 0)),
            pl.BlockSpec((BLK, 1), lambda ph, j: (j, 0)),
            pl.BlockSpec((1, F), lambda ph, j: (0, 0)),
            pl.BlockSpec((1, F), lambda ph, j: (0, 0)),
            pl.BlockSpec((1, F), lambda ph, j: (0, 0)),
            pl.BlockSpec((F, F), lambda ph, j: (0, 0)),
        ],
        out_specs=[
            pl.BlockSpec((BLK, F), lambda ph, j: (j, 0)),
            pl.BlockSpec((BLK, FH), lambda ph, j: (j, 0)),
            pl.BlockSpec((BLK, FH), lambda ph, j: (j, 0)),
        ],
        out_shape=[
            jax.ShapeDtypeStruct((N, F), jnp.float32),
            jax.ShapeDtypeStruct((N, FH), jnp.float32),
            jax.ShapeDtypeStruct((N, FH), jnp.float32),
        ],
        scratch_shapes=[pltpu.VMEM((8, 128), jnp.float32)],
    )


def _conn_body(adj_ref, cx_ref, w1r_ref, b1_ref, w1t_ref, a1_ref,
               w2r_ref, b2_ref, w2t_ref, a2_ref,
               w3r_ref, b3_ref, w3t_ref, a3_ref, out_ref):
    a = adj_ref[0]
    x = cx_ref[0]

    def prelu(z, al):
        return jnp.where(z >= 0, z, al * z)

    t = jnp.dot(a, x, preferred_element_type=jnp.float32)
    h = (jnp.dot(t, w1r_ref[...], preferred_element_type=jnp.float32)
         + b1_ref[...]
         + jnp.dot(x, w1t_ref[...], preferred_element_type=jnp.float32))
    h = prelu(h, a1_ref[...])
    t = jnp.dot(a, h, preferred_element_type=jnp.float32)
    h = (jnp.dot(t, w2r_ref[...], preferred_element_type=jnp.float32)
         + b2_ref[...]
         + jnp.dot(h, w2t_ref[...], preferred_element_type=jnp.float32))
    h = prelu(h, a2_ref[...])
    t = jnp.dot(a, h, preferred_element_type=jnp.float32)
    h = (jnp.dot(t, w3r_ref[...], preferred_element_type=jnp.float32)
         + b3_ref[...]
         + jnp.dot(h, w3t_ref[...], preferred_element_type=jnp.float32))
    h = prelu(h, a3_ref[...])
    out_ref[0] = h


def _make_conn():
    full = lambda r, c: pl.BlockSpec((r, c), lambda g: (0, 0))
    return pl.pallas_call(
        _conn_body,
        grid=(G,),
        in_specs=[
            pl.BlockSpec((1, 87, 87), lambda g: (g, 0, 0)),
            pl.BlockSpec((1, 87, 10), lambda g: (g, 0, 0)),
            full(10, 20), full(1, 20), full(10, 20), full(1, 1),
            full(20, 20), full(1, 20), full(20, 20), full(1, 1),
            full(20, 5), full(1, 5), full(20, 5), full(1, 1),
        ],
        out_specs=pl.BlockSpec((1, 87, 5), lambda g: (g, 0, 0)),
        out_shape=jax.ShapeDtypeStruct((G, 87, 5), jnp.float32),
    )


def _final_body(p_ref, h_ref, dis_ref, fb_ref, g_ref, b_ref, batch_ref,
                conn_ref, w1a_ref, w1b_ref, hb1_ref, hg_ref, hbb_ref,
                w2_ref, hb2_ref, out_ref, acc_ref, seg_ref, cnt_ref):
    j = pl.program_id(0)
    p = p_ref[...]
    h = h_ref[...]
    dis = dis_ref[...]
    u = dis * p + h * (dis * dis) + fb_ref[...]
    u = jnp.maximum(u, 0.0)

    @pl.when(j == 0)
    def _():
        acc_ref[...] = jnp.zeros_like(acc_ref)
        seg_ref[...] = jnp.zeros_like(seg_ref)
        cnt_ref[...] = jnp.zeros_like(cnt_ref)

    acc_ref[0:1, 0:F] += jnp.sum(u, axis=0, keepdims=True)
    acc_ref[1:2, 0:F] += jnp.sum(u * u, axis=0, keepdims=True)

    b = batch_ref[0]
    mask = (lax.broadcasted_iota(jnp.int32, (G, BLK), 0) == b
            ).astype(jnp.float32)
    seg_ref[...] += jnp.dot(mask, u, preferred_element_type=jnp.float32)
    cnt_ref[...] += jnp.sum(mask, axis=1, keepdims=True)

    m = acc_ref[0:1, 0:F] / N
    v = acc_ref[1:2, 0:F] / N - m * m
    rstd = lax.rsqrt(v + 1e-5)
    cnt = jnp.maximum(cnt_ref[...], 1.0)
    mean_u = seg_ref[...] / cnt
    mesh_feat = (mean_u - m) * rstd * g_ref[...] + b_ref[...]
    z = (jnp.dot(mesh_feat, w1a_ref[...], preferred_element_type=jnp.float32)
         + jnp.dot(conn_ref[...], w1b_ref[...],
                   preferred_element_type=jnp.float32)
         + hb1_ref[...])
    z = jnp.maximum(z, 0.0)
    zm = jnp.mean(z, axis=0, keepdims=True)
    zv = jnp.mean(z * z, axis=0, keepdims=True) - zm * zm
    zn = (z - zm) * lax.rsqrt(zv + 1e-5) * hg_ref[...] + hbb_ref[...]
    out_ref[...] = (jnp.dot(zn, w2_ref[...], preferred_element_type=jnp.float32)
                    + hb2_ref[...])


def _make_final():
    full = lambda r, c: pl.BlockSpec((r, c), lambda j: (0, 0))
    return pl.pallas_call(
        _final_body,
        grid=(NB,),
        in_specs=[
            pl.BlockSpec((BLK, F), lambda j: (j, 0)),
            pl.BlockSpec((BLK, F), lambda j: (j, 0)),
            pl.BlockSpec((BLK, 1), lambda j: (j, 0)),
            full(1, F), full(1, F), full(1, F),
            pl.BlockSpec((1, 1, BLK), lambda j: (j, 0, 0)),
            full(G, 435), full(F, 10), full(435, 10), full(1, 10),
            full(1, 10), full(1, 10), full(10, 1), full(1, 1),
        ],
        out_specs=pl.BlockSpec((G, 1), lambda j: (0, 0)),
        out_shape=jax.ShapeDtypeStruct((G, 1), jnp.float32),
        scratch_shapes=[
            pltpu.VMEM((8, 128), jnp.float32),
            pltpu.VMEM((G, F), jnp.float32),
            pltpu.VMEM((G, 1), jnp.float32),
        ],
    )


_k1 = _make_k1()
_klayer = _make_layer()
_kconn = _make_conn()
_kfinal = _make_final()


def kernel(mesh_pos, mesh_norm, mesh_dha, mesh_x, mesh_edge_index, mesh_batch,
           conn_x, conn_adj,
           cw1_rel, cb1, cw1_root, ca1,
           cw2_rel, cb2, cw2_root, ca2,
           cw3_rel, cb3, cw3_root, ca3,
           bn0_g, bn0_b,
           fw1, fb1, bn1_g, bn1_b,
           fw2, fb2, bn2_g, bn2_b,
           fw3, fb3, bn3_g, bn3_b,
           fw4, fb4, bn4_g, bn4_b,
           hw1, hb1, hbn_g, hbn_b, hw2, hb2):
    r = lambda a: a.reshape(1, -1)
    x13 = jnp.concatenate([mesh_pos, mesh_norm, mesh_dha, mesh_x], axis=1)
    src = mesh_edge_index[0]
    dst = mesh_edge_index[1]
    zeros_h = jnp.zeros((NPAD, FH), jnp.float32)
    ones_h = jnp.ones((N, FH), jnp.float32)

    degp = _make_agg()(ones_h, ones_h, dst, dst, zeros_h)

    conn3 = _kconn(conn_adj, conn_x,
                   cw1_rel, r(cb1), cw1_root, r(ca1),
                   cw2_rel, r(cb2), cw2_root, r(ca2),
                   cw3_rel, r(cb3), cw3_root, r(ca3))
    conn_feat = conn3.reshape(G, 435)

    h, hs0, hs1, dis = _k1(x13, degp, r(bn0_g), r(bn0_b), fw1)

    fbs = (fb1, fb2, fb3)
    gs = (bn1_g, bn2_g, bn3_g)
    bs = (bn1_b, bn2_b, bn3_b)
    ws = (fw2, fw3, fw4)
    for i in range(3):
        p = _make_agg()(hs0, hs1, src, dst, zeros_h)
        h, hs0, hs1 = _klayer(p, h, dis, r(fbs[i]), r(gs[i]), r(bs[i]), ws[i])

    p = _make_agg()(hs0, hs1, src, dst, zeros_h)
    out = _kfinal(p, h, dis, r(fb4), r(bn4_g), r(bn4_b),
                  mesh_batch.reshape(NB, 1, BLK), conn_feat,
                  hw1[:F], hw1[F:], r(hb1), r(hbn_g), r(hbn_b), hw2, r(hb2))
    return out


# speedup vs baseline: 29.7732x; 1.5717x over previous
import functools

import jax
import jax.numpy as jnp
from jax import lax
from jax.experimental import pallas as pl
from jax.experimental.pallas import tpu as pltpu
from jax.experimental.pallas import tpu_sc as plsc

N = 50000
E = 1600000
G = 64
F = 32
NC, NS = 2, 16
NW = NC * NS
EPW = E // NW
CH = 2000
NPAD = 50048
ROWS_W = NPAD // NS
DEGW = 16

BLK = 1000
NB = N // BLK


FH = F // NC
EPS = E // NS
NCHS = EPS // CH


def _agg_body(hs0_hbm, hs1_hbm, src_hbm, dst_hbm, zeros_hbm, out_hbm,
              sidx0, sidx1, didx0, didx1, rows0, rows1, agg_sh, sem0, sem1):
    c = lax.axis_index("c")
    s = lax.axis_index("s")
    sidx = (sidx0, sidx1)
    didx = (didx0, didx1)
    rows = (rows0, rows1)
    sems = (sem0, sem1)

    def load_idx(i, b):
        base = s * EPS + i * CH
        pltpu.sync_copy(src_hbm.at[pl.ds(base, CH)], sidx[b])
        pltpu.sync_copy(dst_hbm.at[pl.ds(base, CH)], didx[b])

    def start_gather(b):
        @pl.when(c == 0)
        def _():
            pltpu.async_copy(hs0_hbm.at[sidx[b]], rows[b], sems[b])

        @pl.when(c == 1)
        def _():
            pltpu.async_copy(hs1_hbm.at[sidx[b]], rows[b], sems[b])

    def wait_gather(b):
        pltpu.make_async_copy(hs0_hbm.at[sidx[b]], rows[b], sems[b]).wait()

    pltpu.sync_copy(zeros_hbm.at[pl.ds(s * ROWS_W, ROWS_W)],
                    agg_sh.at[pl.ds(s * ROWS_W, ROWS_W)])
    plsc.subcore_barrier()

    for b in range(2):
        load_idx(b, b)
        start_gather(b)

    def body(k, carry):
        for b in range(2):
            i = k * 2 + b
            wait_gather(b)
            pltpu.sync_copy(rows[b], agg_sh.at[didx[b]], add=True)

            @pl.when(i + 2 < NCHS)
            def _():
                load_idx(i + 2, b)
                start_gather(b)
        return carry

    lax.fori_loop(0, NCHS // 2, body, 0)
    plsc.subcore_barrier()
    pltpu.sync_copy(agg_sh.at[pl.ds(s * ROWS_W, ROWS_W)],
                    out_hbm.at[pl.ds(s * ROWS_W, ROWS_W), pl.ds(c * FH, FH)])


@functools.lru_cache(maxsize=None)
def _make_agg():
    return pl.kernel(
        _agg_body,
        out_type=jax.ShapeDtypeStruct((NPAD, F), jnp.float32),
        mesh=plsc.VectorSubcoreMesh(core_axis_name="c", subcore_axis_name="s"),
        scratch_types=[
            pltpu.VMEM((CH,), jnp.int32),
            pltpu.VMEM((CH,), jnp.int32),
            pltpu.VMEM((CH,), jnp.int32),
            pltpu.VMEM((CH,), jnp.int32),
            pltpu.VMEM((CH, FH), jnp.float32),
            pltpu.VMEM((CH, FH), jnp.float32),
            pltpu.VMEM_SHARED((NPAD, FH), jnp.float32),
            pltpu.SemaphoreType.DMA,
            pltpu.SemaphoreType.DMA,
        ],
        compiler_params=pltpu.CompilerParams(use_tc_tiling_on_sc=False),
    )


def _k1_body(x_ref, degp_ref, g0_ref, b0_ref, w_ref,
             h_ref, hs0_ref, hs1_ref, dis_ref, acc_ref):
    ph = pl.program_id(0)
    j = pl.program_id(1)
    x = x_ref[...]
    dp = degp_ref[...]
    deg = 1.0 + dp[:, 0:1]
    dis = lax.rsqrt(deg)
    dis_ref[...] = dis

    @pl.when(jnp.logical_and(ph == 0, j == 0))
    def _():
        acc_ref[...] = jnp.zeros_like(acc_ref)

    @pl.when(ph == 0)
    def _():
        acc_ref[0:1, 0:13] += jnp.sum(x, axis=0, keepdims=True)
        acc_ref[1:2, 0:13] += jnp.sum(x * x, axis=0, keepdims=True)

    m = acc_ref[0:1, 0:13] / N
    v = acc_ref[1:2, 0:13] / N - m * m
    xn = (x - m) * lax.rsqrt(v + 1e-5) * g0_ref[...] + b0_ref[...]
    h = jnp.dot(xn, w_ref[...], preferred_element_type=jnp.float32)
    h_ref[...] = h
    hsc = h * dis
    hs0_ref[...] = hsc[:, :FH]
    hs1_ref[...] = hsc[:, FH:]


def _make_k1():
    return pl.pallas_call(
        _k1_body,
        grid=(2, NB),
        in_specs=[
            pl.BlockSpec((BLK, 13), lambda ph, j: (j, 0)),
            pl.BlockSpec((BLK, F), lambda ph, j: (j, 0)),
            pl.BlockSpec((1, 13), lambda ph, j: (0, 0)),
            pl.BlockSpec((1, 13), lambda ph, j: (0, 0)),
            pl.BlockSpec((13, F), lambda ph, j: (0, 0)),
        ],
        out_specs=[
            pl.BlockSpec((BLK, F), lambda ph, j: (j, 0)),
            pl.BlockSpec((BLK, FH), lambda ph, j: (j, 0)),
            pl.BlockSpec((BLK, FH), lambda ph, j: (j, 0)),
            pl.BlockSpec((BLK, 1), lambda ph, j: (j, 0)),
        ],
        out_shape=[
            jax.ShapeDtypeStruct((N, F), jnp.float32),
            jax.ShapeDtypeStruct((N, FH), jnp.float32),
            jax.ShapeDtypeStruct((N, FH), jnp.float32),
            jax.ShapeDtypeStruct((N, 1), jnp.float32),
        ],
        scratch_shapes=[pltpu.VMEM((8, 128), jnp.float32)],
    )


def _layer_body(p_ref, h_ref, dis_ref, fb_ref, g_ref, b_ref, w_ref,
                hn_ref, hs0_ref, hs1_ref, acc_ref):
    ph = pl.program_id(0)
    j = pl.program_id(1)
    p = p_ref[...]
    h = h_ref[...]
    dis = dis_ref[...]
    u = dis * p + h * (dis * dis) + fb_ref[...]
    u = jnp.maximum(u, 0.0)

    @pl.when(jnp.logical_and(ph == 0, j == 0))
    def _():
        acc_ref[...] = jnp.zeros_like(acc_ref)

    @pl.when(ph == 0)
    def _():
        acc_ref[0:1, 0:F] += jnp.sum(u, axis=0, keepdims=True)
        acc_ref[1:2, 0:F] += jnp.sum(u * u, axis=0, keepdims=True)

    m = acc_ref[0:1, 0:F] / N
    v = acc_ref[1:2, 0:F] / N - m * m
    mx = (u - m) * lax.rsqrt(v + 1e-5) * g_ref[...] + b_ref[...]
    hn = jnp.dot(mx, w_ref[...], preferred_element_type=jnp.float32)
    hn_ref[...] = hn
    hsc = hn * dis
    hs0_ref[...] = hsc[:, :FH]
    hs1_ref[...] = hsc[:, FH:]


def _make_layer():
    return pl.pallas_call(
        _layer_body,
        grid=(2, NB),
        in_specs=[
            pl.BlockSpec((BLK, F), lambda ph, j: (j, 0)),
            pl.BlockSpec((BLK, F), lambda ph, j: (j, 0)),
            pl.BlockSpec((BLK, 1), lambda ph, j: (j, 0)),
            pl.BlockSpec((1, F), lambda ph, j: (0, 0)),
            pl.BlockSpec((1, F), lambda ph, j: (0, 0)),
            pl.BlockSpec((1, F), lambda ph, j: (0, 0)),
            pl.BlockSpec((F, F), lambda ph, j: (0, 0)),
        ],
        out_specs=[
            pl.BlockSpec((BLK, F), lambda ph, j: (j, 0)),
            pl.BlockSpec((BLK, FH), lambda ph, j: (j, 0)),
            pl.BlockSpec((BLK, FH), lambda ph, j: (j, 0)),
        ],
        out_shape=[
            jax.ShapeDtypeStruct((N, F), jnp.float32),
            jax.ShapeDtypeStruct((N, FH), jnp.float32),
            jax.ShapeDtypeStruct((N, FH), jnp.float32),
        ],
        scratch_shapes=[pltpu.VMEM((8, 128), jnp.float32)],
    )


def _conn_body(adj_ref, cx_ref, w1r_ref, b1_ref, w1t_ref, a1_ref,
               w2r_ref, b2_ref, w2t_ref, a2_ref,
               w3r_ref, b3_ref, w3t_ref, a3_ref, out_ref):
    a = adj_ref[0]
    x = cx_ref[0]

    def prelu(z, al):
        return jnp.where(z >= 0, z, al * z)

    t = jnp.dot(a, x, preferred_element_type=jnp.float32)
    h = (jnp.dot(t, w1r_ref[...], preferred_element_type=jnp.float32)
         + b1_ref[...]
         + jnp.dot(x, w1t_ref[...], preferred_element_type=jnp.float32))
    h = prelu(h, a1_ref[...])
    t = jnp.dot(a, h, preferred_element_type=jnp.float32)
    h = (jnp.dot(t, w2r_ref[...], preferred_element_type=jnp.float32)
         + b2_ref[...]
         + jnp.dot(h, w2t_ref[...], preferred_element_type=jnp.float32))
    h = prelu(h, a2_ref[...])
    t = jnp.dot(a, h, preferred_element_type=jnp.float32)
    h = (jnp.dot(t, w3r_ref[...], preferred_element_type=jnp.float32)
         + b3_ref[...]
         + jnp.dot(h, w3t_ref[...], preferred_element_type=jnp.float32))
    h = prelu(h, a3_ref[...])
    out_ref[0] = h


def _make_conn():
    full = lambda r, c: pl.BlockSpec((r, c), lambda g: (0, 0))
    return pl.pallas_call(
        _conn_body,
        grid=(G,),
        in_specs=[
            pl.BlockSpec((1, 87, 87), lambda g: (g, 0, 0)),
            pl.BlockSpec((1, 87, 10), lambda g: (g, 0, 0)),
            full(10, 20), full(1, 20), full(10, 20), full(1, 1),
            full(20, 20), full(1, 20), full(20, 20), full(1, 1),
            full(20, 5), full(1, 5), full(20, 5), full(1, 1),
        ],
        out_specs=pl.BlockSpec((1, 87, 5), lambda g: (g, 0, 0)),
        out_shape=jax.ShapeDtypeStruct((G, 87, 5), jnp.float32),
    )


def _final_body(p_ref, h_ref, dis_ref, fb_ref, g_ref, b_ref, batch_ref,
                conn_ref, w1a_ref, w1b_ref, hb1_ref, hg_ref, hbb_ref,
                w2_ref, hb2_ref, out_ref, acc_ref, seg_ref, cnt_ref):
    j = pl.program_id(0)
    p = p_ref[...]
    h = h_ref[...]
    dis = dis_ref[...]
    u = dis * p + h * (dis * dis) + fb_ref[...]
    u = jnp.maximum(u, 0.0)

    @pl.when(j == 0)
    def _():
        acc_ref[...] = jnp.zeros_like(acc_ref)
        seg_ref[...] = jnp.zeros_like(seg_ref)
        cnt_ref[...] = jnp.zeros_like(cnt_ref)

    acc_ref[0:1, 0:F] += jnp.sum(u, axis=0, keepdims=True)
    acc_ref[1:2, 0:F] += jnp.sum(u * u, axis=0, keepdims=True)

    b = batch_ref[0]
    mask = (lax.broadcasted_iota(jnp.int32, (G, BLK), 0) == b
            ).astype(jnp.float32)
    seg_ref[...] += jnp.dot(mask, u, preferred_element_type=jnp.float32)
    cnt_ref[...] += jnp.sum(mask, axis=1, keepdims=True)

    m = acc_ref[0:1, 0:F] / N
    v = acc_ref[1:2, 0:F] / N - m * m
    rstd = lax.rsqrt(v + 1e-5)
    cnt = jnp.maximum(cnt_ref[...], 1.0)
    mean_u = seg_ref[...] / cnt
    mesh_feat = (mean_u - m) * rstd * g_ref[...] + b_ref[...]
    z = (jnp.dot(mesh_feat, w1a_ref[...], preferred_element_type=jnp.float32)
         + jnp.dot(conn_ref[...], w1b_ref[...],
                   preferred_element_type=jnp.float32)
         + hb1_ref[...])
    z = jnp.maximum(z, 0.0)
    zm = jnp.mean(z, axis=0, keepdims=True)
    zv = jnp.mean(z * z, axis=0, keepdims=True) - zm * zm
    zn = (z - zm) * lax.rsqrt(zv + 1e-5) * hg_ref[...] + hbb_ref[...]
    out_ref[...] = (jnp.dot(zn, w2_ref[...], preferred_element_type=jnp.float32)
                    + hb2_ref[...])


def _make_final():
    full = lambda r, c: pl.BlockSpec((r, c), lambda j: (0, 0))
    return pl.pallas_call(
        _final_body,
        grid=(NB,),
        in_specs=[
            pl.BlockSpec((BLK, F), lambda j: (j, 0)),
            pl.BlockSpec((BLK, F), lambda j: (j, 0)),
            pl.BlockSpec((BLK, 1), lambda j: (j, 0)),
            full(1, F), full(1, F), full(1, F),
            pl.BlockSpec((1, 1, BLK), lambda j: (j, 0, 0)),
            full(G, 435), full(F, 10), full(435, 10), full(1, 10),
            full(1, 10), full(1, 10), full(10, 1), full(1, 1),
        ],
        out_specs=pl.BlockSpec((G, 1), lambda j: (0, 0)),
        out_shape=jax.ShapeDtypeStruct((G, 1), jnp.float32),
        scratch_shapes=[
            pltpu.VMEM((8, 128), jnp.float32),
            pltpu.VMEM((G, F), jnp.float32),
            pltpu.VMEM((G, 1), jnp.float32),
        ],
    )


_k1 = _make_k1()
_klayer = _make_layer()
_kconn = _make_conn()
_kfinal = _make_final()


def kernel(mesh_pos, mesh_norm, mesh_dha, mesh_x, mesh_edge_index, mesh_batch,
           conn_x, conn_adj,
           cw1_rel, cb1, cw1_root, ca1,
           cw2_rel, cb2, cw2_root, ca2,
           cw3_rel, cb3, cw3_root, ca3,
           bn0_g, bn0_b,
           fw1, fb1, bn1_g, bn1_b,
           fw2, fb2, bn2_g, bn2_b,
           fw3, fb3, bn3_g, bn3_b,
           fw4, fb4, bn4_g, bn4_b,
           hw1, hb1, hbn_g, hbn_b, hw2, hb2):
    r = lambda a: a.reshape(1, -1)
    x13 = jnp.concatenate([mesh_pos, mesh_norm, mesh_dha, mesh_x], axis=1)
    src = mesh_edge_index[0]
    dst = mesh_edge_index[1]
    zeros_h = jnp.zeros((NPAD, FH), jnp.float32)
    ones_h = jnp.ones((N, FH), jnp.float32)

    degp = _make_agg()(ones_h, ones_h, dst, dst, zeros_h)

    conn3 = _kconn(conn_adj, conn_x,
                   cw1_rel, r(cb1), cw1_root, r(ca1),
                   cw2_rel, r(cb2), cw2_root, r(ca2),
                   cw3_rel, r(cb3), cw3_root, r(ca3))
    conn_feat = conn3.reshape(G, 435)

    h, hs0, hs1, dis = _k1(x13, degp, r(bn0_g), r(bn0_b), fw1)

    fbs = (fb1, fb2, fb3)
    gs = (bn1_g, bn2_g, bn3_g)
    bs = (bn1_b, bn2_b, bn3_b)
    ws = (fw2, fw3, fw4)
    for i in range(3):
        p = _make_agg()(hs0, hs1, src, dst, zeros_h)
        h, hs0, hs1 = _klayer(p, h, dis, r(fbs[i]), r(gs[i]), r(bs[i]), ws[i])

    p = _make_agg()(hs0, hs1, src, dst, zeros_h)
    out = _kfinal(p, h, dis, r(fb4), r(bn4_g), r(bn4_b),
                  mesh_batch.reshape(NB, 1, BLK), conn_feat,
                  hw1[:F], hw1[F:], r(hb1), r(hbn_g), r(hbn_b), hw2, r(hb2))
    return out


# speedup vs baseline: 32.3664x; 1.0871x over previous
import functools

import jax
import jax.numpy as jnp
from jax import lax
from jax.experimental import pallas as pl
from jax.experimental.pallas import tpu as pltpu
from jax.experimental.pallas import tpu_sc as plsc

N = 50000
E = 1600000
G = 64
F = 32
NC, NS = 2, 16
NW = NC * NS
EPW = E // NW
CH = 2000
NPAD = 50048
ROWS_W = NPAD // NS
DEGW = 16

BLK = 2000
NB = N // BLK


FH = F // NC
EPS = E // NS
NCHS = EPS // CH


def _agg_body(hs0_hbm, hs1_hbm, src_hbm, dst_hbm, zeros_hbm, out_hbm,
              sidx0, sidx1, didx0, didx1, rows0, rows1, agg_sh, sem0, sem1):
    c = lax.axis_index("c")
    s = lax.axis_index("s")
    sidx = (sidx0, sidx1)
    didx = (didx0, didx1)
    rows = (rows0, rows1)
    sems = (sem0, sem1)

    def load_idx(i, b):
        base = s * EPS + i * CH
        pltpu.sync_copy(src_hbm.at[pl.ds(base, CH)], sidx[b])
        pltpu.sync_copy(dst_hbm.at[pl.ds(base, CH)], didx[b])

    def start_gather(b):
        @pl.when(c == 0)
        def _():
            pltpu.async_copy(hs0_hbm.at[sidx[b]], rows[b], sems[b])

        @pl.when(c == 1)
        def _():
            pltpu.async_copy(hs1_hbm.at[sidx[b]], rows[b], sems[b])

    def wait_gather(b):
        pltpu.make_async_copy(hs0_hbm.at[sidx[b]], rows[b], sems[b]).wait()

    pltpu.sync_copy(zeros_hbm.at[pl.ds(s * ROWS_W, ROWS_W)],
                    agg_sh.at[pl.ds(s * ROWS_W, ROWS_W)])
    plsc.subcore_barrier()

    for b in range(2):
        load_idx(b, b)
        start_gather(b)

    def body(k, carry):
        for b in range(2):
            i = k * 2 + b
            wait_gather(b)
            pltpu.sync_copy(rows[b], agg_sh.at[didx[b]], add=True)

            @pl.when(i + 2 < NCHS)
            def _():
                load_idx(i + 2, b)
                start_gather(b)
        return carry

    lax.fori_loop(0, NCHS // 2, body, 0)
    plsc.subcore_barrier()
    pltpu.sync_copy(agg_sh.at[pl.ds(s * ROWS_W, ROWS_W)],
                    out_hbm.at[pl.ds(s * ROWS_W, ROWS_W), pl.ds(c * FH, FH)])


@functools.lru_cache(maxsize=None)
def _make_agg():
    return pl.kernel(
        _agg_body,
        out_type=jax.ShapeDtypeStruct((NPAD, F), jnp.float32),
        mesh=plsc.VectorSubcoreMesh(core_axis_name="c", subcore_axis_name="s"),
        scratch_types=[
            pltpu.VMEM((CH,), jnp.int32),
            pltpu.VMEM((CH,), jnp.int32),
            pltpu.VMEM((CH,), jnp.int32),
            pltpu.VMEM((CH,), jnp.int32),
            pltpu.VMEM((CH, FH), jnp.float32),
            pltpu.VMEM((CH, FH), jnp.float32),
            pltpu.VMEM_SHARED((NPAD, FH), jnp.float32),
            pltpu.SemaphoreType.DMA,
            pltpu.SemaphoreType.DMA,
        ],
        compiler_params=pltpu.CompilerParams(use_tc_tiling_on_sc=False),
    )


def _k1_body(x_ref, degp_ref, g0_ref, b0_ref, w_ref,
             h_ref, hs0_ref, hs1_ref, dis_ref, acc_ref):
    ph = pl.program_id(0)
    j = pl.program_id(1)
    x = x_ref[...]
    dp = degp_ref[...]
    deg = 1.0 + dp[:, 0:1]
    dis = lax.rsqrt(deg)
    dis_ref[...] = dis

    @pl.when(jnp.logical_and(ph == 0, j == 0))
    def _():
        acc_ref[...] = jnp.zeros_like(acc_ref)

    @pl.when(ph == 0)
    def _():
        acc_ref[0:1, 0:13] += jnp.sum(x, axis=0, keepdims=True)
        acc_ref[1:2, 0:13] += jnp.sum(x * x, axis=0, keepdims=True)

    m = acc_ref[0:1, 0:13] / N
    v = acc_ref[1:2, 0:13] / N - m * m
    xn = (x - m) * lax.rsqrt(v + 1e-5) * g0_ref[...] + b0_ref[...]
    h = jnp.dot(xn, w_ref[...], preferred_element_type=jnp.float32)
    h_ref[...] = h
    hsc = h * dis
    hs0_ref[...] = hsc[:, :FH]
    hs1_ref[...] = hsc[:, FH:]


def _make_k1():
    return pl.pallas_call(
        _k1_body,
        grid=(2, NB),
        in_specs=[
            pl.BlockSpec((BLK, 13), lambda ph, j: (j, 0)),
            pl.BlockSpec((BLK, F), lambda ph, j: (j, 0)),
            pl.BlockSpec((1, 13), lambda ph, j: (0, 0)),
            pl.BlockSpec((1, 13), lambda ph, j: (0, 0)),
            pl.BlockSpec((13, F), lambda ph, j: (0, 0)),
        ],
        out_specs=[
            pl.BlockSpec((BLK, F), lambda ph, j: (j, 0)),
            pl.BlockSpec((BLK, FH), lambda ph, j: (j, 0)),
            pl.BlockSpec((BLK, FH), lambda ph, j: (j, 0)),
            pl.BlockSpec((BLK, 1), lambda ph, j: (j, 0)),
        ],
        out_shape=[
            jax.ShapeDtypeStruct((N, F), jnp.float32),
            jax.ShapeDtypeStruct((N, FH), jnp.float32),
            jax.ShapeDtypeStruct((N, FH), jnp.float32),
            jax.ShapeDtypeStruct((N, 1), jnp.float32),
        ],
        scratch_shapes=[pltpu.VMEM((8, 128), jnp.float32)],
    )


def _layer_body(p_ref, h_ref, dis_ref, fb_ref, g_ref, b_ref, w_ref,
                hn_ref, hs0_ref, hs1_ref, acc_ref):
    ph = pl.program_id(0)
    j = pl.program_id(1)
    p = p_ref[...]
    h = h_ref[...]
    dis = dis_ref[...]
    u = dis * p + h * (dis * dis) + fb_ref[...]
    u = jnp.maximum(u, 0.0)

    @pl.when(jnp.logical_and(ph == 0, j == 0))
    def _():
        acc_ref[...] = jnp.zeros_like(acc_ref)

    @pl.when(ph == 0)
    def _():
        acc_ref[0:1, 0:F] += jnp.sum(u, axis=0, keepdims=True)
        acc_ref[1:2, 0:F] += jnp.sum(u * u, axis=0, keepdims=True)

    m = acc_ref[0:1, 0:F] / N
    v = acc_ref[1:2, 0:F] / N - m * m
    mx = (u - m) * lax.rsqrt(v + 1e-5) * g_ref[...] + b_ref[...]
    hn = jnp.dot(mx, w_ref[...], preferred_element_type=jnp.float32)
    hn_ref[...] = hn
    hsc = hn * dis
    hs0_ref[...] = hsc[:, :FH]
    hs1_ref[...] = hsc[:, FH:]


def _make_layer():
    return pl.pallas_call(
        _layer_body,
        grid=(2, NB),
        in_specs=[
            pl.BlockSpec((BLK, F), lambda ph, j: (j, 0)),
            pl.BlockSpec((BLK, F), lambda ph, j: (j, 0)),
            pl.BlockSpec((BLK, 1), lambda ph, j: (j, 0)),
            pl.BlockSpec((1, F), lambda ph, j: (0, 0)),
            pl.BlockSpec((1, F), lambda ph, j: (0, 0)),
            pl.BlockSpec((1, F), lambda ph, j: (0, 0)),
            pl.BlockSpec((F, F), lambda ph, j: (0, 0)),
        ],
        out_specs=[
            pl.BlockSpec((BLK, F), lambda ph, j: (j, 0)),
            pl.BlockSpec((BLK, FH), lambda ph, j: (j, 0)),
            pl.BlockSpec((BLK, FH), lambda ph, j: (j, 0)),
        ],
        out_shape=[
            jax.ShapeDtypeStruct((N, F), jnp.float32),
            jax.ShapeDtypeStruct((N, FH), jnp.float32),
            jax.ShapeDtypeStruct((N, FH), jnp.float32),
        ],
        scratch_shapes=[pltpu.VMEM((8, 128), jnp.float32)],
    )


def _conn_body(adj_ref, cx_ref, w1r_ref, b1_ref, w1t_ref, a1_ref,
               w2r_ref, b2_ref, w2t_ref, a2_ref,
               w3r_ref, b3_ref, w3t_ref, a3_ref, out_ref):
    a = adj_ref[0]
    x = cx_ref[0]

    def prelu(z, al):
        return jnp.where(z >= 0, z, al * z)

    t = jnp.dot(a, x, preferred_element_type=jnp.float32)
    h = (jnp.dot(t, w1r_ref[...], preferred_element_type=jnp.float32)
         + b1_ref[...]
         + jnp.dot(x, w1t_ref[...], preferred_element_type=jnp.float32))
    h = prelu(h, a1_ref[...])
    t = jnp.dot(a, h, preferred_element_type=jnp.float32)
    h = (jnp.dot(t, w2r_ref[...], preferred_element_type=jnp.float32)
         + b2_ref[...]
         + jnp.dot(h, w2t_ref[...], preferred_element_type=jnp.float32))
    h = prelu(h, a2_ref[...])
    t = jnp.dot(a, h, preferred_element_type=jnp.float32)
    h = (jnp.dot(t, w3r_ref[...], preferred_element_type=jnp.float32)
         + b3_ref[...]
         + jnp.dot(h, w3t_ref[...], preferred_element_type=jnp.float32))
    h = prelu(h, a3_ref[...])
    out_ref[0] = h


def _make_conn():
    full = lambda r, c: pl.BlockSpec((r, c), lambda g: (0, 0))
    return pl.pallas_call(
        _conn_body,
        grid=(G,),
        in_specs=[
            pl.BlockSpec((1, 87, 87), lambda g: (g, 0, 0)),
            pl.BlockSpec((1, 87, 10), lambda g: (g, 0, 0)),
            full(10, 20), full(1, 20), full(10, 20), full(1, 1),
            full(20, 20), full(1, 20), full(20, 20), full(1, 1),
            full(20, 5), full(1, 5), full(20, 5), full(1, 1),
        ],
        out_specs=pl.BlockSpec((1, 87, 5), lambda g: (g, 0, 0)),
        out_shape=jax.ShapeDtypeStruct((G, 87, 5), jnp.float32),
    )


def _final_body(p_ref, h_ref, dis_ref, fb_ref, g_ref, b_ref, batch_ref,
                conn_ref, w1a_ref, w1b_ref, hb1_ref, hg_ref, hbb_ref,
                w2_ref, hb2_ref, out_ref, acc_ref, seg_ref, cnt_ref):
    j = pl.program_id(0)
    p = p_ref[...]
    h = h_ref[...]
    dis = dis_ref[...]
    u = dis * p + h * (dis * dis) + fb_ref[...]
    u = jnp.maximum(u, 0.0)

    @pl.when(j == 0)
    def _():
        acc_ref[...] = jnp.zeros_like(acc_ref)
        seg_ref[...] = jnp.zeros_like(seg_ref)
        cnt_ref[...] = jnp.zeros_like(cnt_ref)

    acc_ref[0:1, 0:F] += jnp.sum(u, axis=0, keepdims=True)
    acc_ref[1:2, 0:F] += jnp.sum(u * u, axis=0, keepdims=True)

    b = batch_ref[0]
    mask = (lax.broadcasted_iota(jnp.int32, (G, BLK), 0) == b
            ).astype(jnp.float32)
    seg_ref[...] += jnp.dot(mask, u, preferred_element_type=jnp.float32)
    cnt_ref[...] += jnp.sum(mask, axis=1, keepdims=True)

    m = acc_ref[0:1, 0:F] / N
    v = acc_ref[1:2, 0:F] / N - m * m
    rstd = lax.rsqrt(v + 1e-5)
    cnt = jnp.maximum(cnt_ref[...], 1.0)
    mean_u = seg_ref[...] / cnt
    mesh_feat = (mean_u - m) * rstd * g_ref[...] + b_ref[...]
    z = (jnp.dot(mesh_feat, w1a_ref[...], preferred_element_type=jnp.float32)
         + jnp.dot(conn_ref[...], w1b_ref[...],
                   preferred_element_type=jnp.float32)
         + hb1_ref[...])
    z = jnp.maximum(z, 0.0)
    zm = jnp.mean(z, axis=0, keepdims=True)
    zv = jnp.mean(z * z, axis=0, keepdims=True) - zm * zm
    zn = (z - zm) * lax.rsqrt(zv + 1e-5) * hg_ref[...] + hbb_ref[...]
    out_ref[...] = (jnp.dot(zn, w2_ref[...], preferred_element_type=jnp.float32)
                    + hb2_ref[...])


def _make_final():
    full = lambda r, c: pl.BlockSpec((r, c), lambda j: (0, 0))
    return pl.pallas_call(
        _final_body,
        grid=(NB,),
        in_specs=[
            pl.BlockSpec((BLK, F), lambda j: (j, 0)),
            pl.BlockSpec((BLK, F), lambda j: (j, 0)),
            pl.BlockSpec((BLK, 1), lambda j: (j, 0)),
            full(1, F), full(1, F), full(1, F),
            pl.BlockSpec((1, 1, BLK), lambda j: (j, 0, 0)),
            full(G, 435), full(F, 10), full(435, 10), full(1, 10),
            full(1, 10), full(1, 10), full(10, 1), full(1, 1),
        ],
        out_specs=pl.BlockSpec((G, 1), lambda j: (0, 0)),
        out_shape=jax.ShapeDtypeStruct((G, 1), jnp.float32),
        scratch_shapes=[
            pltpu.VMEM((8, 128), jnp.float32),
            pltpu.VMEM((G, F), jnp.float32),
            pltpu.VMEM((G, 1), jnp.float32),
        ],
    )


_k1 = _make_k1()
_klayer = _make_layer()
_kconn = _make_conn()
_kfinal = _make_final()


def kernel(mesh_pos, mesh_norm, mesh_dha, mesh_x, mesh_edge_index, mesh_batch,
           conn_x, conn_adj,
           cw1_rel, cb1, cw1_root, ca1,
           cw2_rel, cb2, cw2_root, ca2,
           cw3_rel, cb3, cw3_root, ca3,
           bn0_g, bn0_b,
           fw1, fb1, bn1_g, bn1_b,
           fw2, fb2, bn2_g, bn2_b,
           fw3, fb3, bn3_g, bn3_b,
           fw4, fb4, bn4_g, bn4_b,
           hw1, hb1, hbn_g, hbn_b, hw2, hb2):
    r = lambda a: a.reshape(1, -1)
    x13 = jnp.concatenate([mesh_pos, mesh_norm, mesh_dha, mesh_x], axis=1)
    src = mesh_edge_index[0]
    dst = mesh_edge_index[1]
    zeros_h = jnp.zeros((NPAD, FH), jnp.float32)
    ones_h = jnp.ones((N, FH), jnp.float32)

    degp = _make_agg()(ones_h, ones_h, dst, dst, zeros_h)

    conn3 = _kconn(conn_adj, conn_x,
                   cw1_rel, r(cb1), cw1_root, r(ca1),
                   cw2_rel, r(cb2), cw2_root, r(ca2),
                   cw3_rel, r(cb3), cw3_root, r(ca3))
    conn_feat = conn3.reshape(G, 435)

    h, hs0, hs1, dis = _k1(x13, degp, r(bn0_g), r(bn0_b), fw1)

    fbs = (fb1, fb2, fb3)
    gs = (bn1_g, bn2_g, bn3_g)
    bs = (bn1_b, bn2_b, bn3_b)
    ws = (fw2, fw3, fw4)
    for i in range(3):
        p = _make_agg()(hs0, hs1, src, dst, zeros_h)
        h, hs0, hs1 = _klayer(p, h, dis, r(fbs[i]), r(gs[i]), r(bs[i]), ws[i])

    p = _make_agg()(hs0, hs1, src, dst, zeros_h)
    out = _kfinal(p, h, dis, r(fb4), r(bn4_g), r(bn4_b),
                  mesh_batch.reshape(NB, 1, BLK), conn_feat,
                  hw1[:F], hw1[F:], r(hb1), r(hbn_g), r(hbn_b), hw2, r(hb2))
    return out


# speedup vs baseline: 34.4276x; 1.0637x over previous
import functools

import jax
import jax.numpy as jnp
from jax import lax
from jax.experimental import pallas as pl
from jax.experimental.pallas import tpu as pltpu
from jax.experimental.pallas import tpu_sc as plsc

N = 50000
E = 1600000
G = 64
F = 32
NC, NS = 2, 16
NW = NC * NS
EPW = E // NW
CH = 2000
NPAD = 50048
ROWS_W = NPAD // NS
DEGW = 16

BLK = 5000
NB = N // BLK


FH = F // NC
EPS = E // NS
NCHS = EPS // CH


def _agg_body(hs0_hbm, hs1_hbm, ei_hbm, zeros_hbm, out_hbm,
              sidx0, sidx1, didx0, didx1, rows0, rows1, agg_sh, sem0, sem1):
    c = lax.axis_index("c")
    s = lax.axis_index("s")
    sidx = (sidx0, sidx1)
    didx = (didx0, didx1)
    rows = (rows0, rows1)
    sems = (sem0, sem1)

    def load_idx(i, b):
        base = s * EPS + i * CH
        pltpu.sync_copy(ei_hbm.at[0, pl.ds(base, CH)], sidx[b])
        pltpu.sync_copy(ei_hbm.at[1, pl.ds(base, CH)], didx[b])

    def start_gather(b):
        @pl.when(c == 0)
        def _():
            pltpu.async_copy(hs0_hbm.at[sidx[b]], rows[b], sems[b])

        @pl.when(c == 1)
        def _():
            pltpu.async_copy(hs1_hbm.at[sidx[b]], rows[b], sems[b])

    def wait_gather(b):
        pltpu.make_async_copy(hs0_hbm.at[sidx[b]], rows[b], sems[b]).wait()

    pltpu.sync_copy(zeros_hbm.at[pl.ds(s * ROWS_W, ROWS_W)],
                    agg_sh.at[pl.ds(s * ROWS_W, ROWS_W)])
    plsc.subcore_barrier()

    for b in range(2):
        load_idx(b, b)
        start_gather(b)

    def body(k, carry):
        for b in range(2):
            i = k * 2 + b
            wait_gather(b)
            pltpu.sync_copy(rows[b], agg_sh.at[didx[b]], add=True)

            @pl.when(i + 2 < NCHS)
            def _():
                load_idx(i + 2, b)
                start_gather(b)
        return carry

    lax.fori_loop(0, NCHS // 2, body, 0)
    plsc.subcore_barrier()
    pltpu.sync_copy(agg_sh.at[pl.ds(s * ROWS_W, ROWS_W)],
                    out_hbm.at[pl.ds(s * ROWS_W, ROWS_W), pl.ds(c * FH, FH)])


@functools.lru_cache(maxsize=None)
def _make_agg():
    return pl.kernel(
        _agg_body,
        out_type=jax.ShapeDtypeStruct((NPAD, F), jnp.float32),
        mesh=plsc.VectorSubcoreMesh(core_axis_name="c", subcore_axis_name="s"),
        scratch_types=[
            pltpu.VMEM((CH,), jnp.int32),
            pltpu.VMEM((CH,), jnp.int32),
            pltpu.VMEM((CH,), jnp.int32),
            pltpu.VMEM((CH,), jnp.int32),
            pltpu.VMEM((CH, FH), jnp.float32),
            pltpu.VMEM((CH, FH), jnp.float32),
            pltpu.VMEM_SHARED((NPAD, FH), jnp.float32),
            pltpu.SemaphoreType.DMA,
            pltpu.SemaphoreType.DMA,
        ],
        compiler_params=pltpu.CompilerParams(use_tc_tiling_on_sc=False),
    )


def _k1_body(x_ref, degp_ref, g0_ref, b0_ref, w_ref,
             h_ref, hs0_ref, hs1_ref, dis_ref, acc_ref):
    ph = pl.program_id(0)
    j = pl.program_id(1)
    x = x_ref[...]
    dp = degp_ref[...]
    deg = 1.0 + dp[:, 0:1]
    dis = lax.rsqrt(deg)
    dis_ref[...] = dis

    @pl.when(jnp.logical_and(ph == 0, j == 0))
    def _():
        acc_ref[...] = jnp.zeros_like(acc_ref)

    @pl.when(ph == 0)
    def _():
        acc_ref[0:1, 0:13] += jnp.sum(x, axis=0, keepdims=True)
        acc_ref[1:2, 0:13] += jnp.sum(x * x, axis=0, keepdims=True)

    m = acc_ref[0:1, 0:13] / N
    v = acc_ref[1:2, 0:13] / N - m * m
    xn = (x - m) * lax.rsqrt(v + 1e-5) * g0_ref[...] + b0_ref[...]
    h = jnp.dot(xn, w_ref[...], preferred_element_type=jnp.float32)
    h_ref[...] = h
    hsc = h * dis
    hs0_ref[...] = hsc[:, :FH]
    hs1_ref[...] = hsc[:, FH:]


def _make_k1():
    return pl.pallas_call(
        _k1_body,
        grid=(2, NB),
        in_specs=[
            pl.BlockSpec((BLK, 13), lambda ph, j: (j, 0)),
            pl.BlockSpec((BLK, F), lambda ph, j: (j, 0)),
            pl.BlockSpec((1, 13), lambda ph, j: (0, 0)),
            pl.BlockSpec((1, 13), lambda ph, j: (0, 0)),
            pl.BlockSpec((13, F), lambda ph, j: (0, 0)),
        ],
        out_specs=[
            pl.BlockSpec((BLK, F), lambda ph, j: (j, 0)),
            pl.BlockSpec((BLK, FH), lambda ph, j: (j, 0)),
            pl.BlockSpec((BLK, FH), lambda ph, j: (j, 0)),
            pl.BlockSpec((BLK, 1), lambda ph, j: (j, 0)),
        ],
        out_shape=[
            jax.ShapeDtypeStruct((N, F), jnp.float32),
            jax.ShapeDtypeStruct((N, FH), jnp.float32),
            jax.ShapeDtypeStruct((N, FH), jnp.float32),
            jax.ShapeDtypeStruct((N, 1), jnp.float32),
        ],
        scratch_shapes=[pltpu.VMEM((8, 128), jnp.float32)],
    )


def _layer_body(p_ref, h_ref, dis_ref, fb_ref, g_ref, b_ref, w_ref,
                hn_ref, hs0_ref, hs1_ref, acc_ref):
    ph = pl.program_id(0)
    j = pl.program_id(1)
    p = p_ref[...]
    h = h_ref[...]
    dis = dis_ref[...]
    u = dis * p + h * (dis * dis) + fb_ref[...]
    u = jnp.maximum(u, 0.0)

    @pl.when(jnp.logical_and(ph == 0, j == 0))
    def _():
        acc_ref[...] = jnp.zeros_like(acc_ref)

    @pl.when(ph == 0)
    def _():
        acc_ref[0:1, 0:F] += jnp.sum(u, axis=0, keepdims=True)
        acc_ref[1:2, 0:F] += jnp.sum(u * u, axis=0, keepdims=True)

    m = acc_ref[0:1, 0:F] / N
    v = acc_ref[1:2, 0:F] / N - m * m
    mx = (u - m) * lax.rsqrt(v + 1e-5) * g_ref[...] + b_ref[...]
    hn = jnp.dot(mx, w_ref[...], preferred_element_type=jnp.float32)
    hn_ref[...] = hn
    hsc = hn * dis
    hs0_ref[...] = hsc[:, :FH]
    hs1_ref[...] = hsc[:, FH:]


def _make_layer():
    return pl.pallas_call(
        _layer_body,
        grid=(2, NB),
        in_specs=[
            pl.BlockSpec((BLK, F), lambda ph, j: (j, 0)),
            pl.BlockSpec((BLK, F), lambda ph, j: (j, 0)),
            pl.BlockSpec((BLK, 1), lambda ph, j: (j, 0)),
            pl.BlockSpec((1, F), lambda ph, j: (0, 0)),
            pl.BlockSpec((1, F), lambda ph, j: (0, 0)),
            pl.BlockSpec((1, F), lambda ph, j: (0, 0)),
            pl.BlockSpec((F, F), lambda ph, j: (0, 0)),
        ],
        out_specs=[
            pl.BlockSpec((BLK, F), lambda ph, j: (j, 0)),
            pl.BlockSpec((BLK, FH), lambda ph, j: (j, 0)),
            pl.BlockSpec((BLK, FH), lambda ph, j: (j, 0)),
        ],
        out_shape=[
            jax.ShapeDtypeStruct((N, F), jnp.float32),
            jax.ShapeDtypeStruct((N, FH), jnp.float32),
            jax.ShapeDtypeStruct((N, FH), jnp.float32),
        ],
        scratch_shapes=[pltpu.VMEM((8, 128), jnp.float32)],
    )


def _conn_body(adj_ref, cx_ref, w1r_ref, b1_ref, w1t_ref, a1_ref,
               w2r_ref, b2_ref, w2t_ref, a2_ref,
               w3r_ref, b3_ref, w3t_ref, a3_ref, out_ref):
    a = adj_ref[0]
    x = cx_ref[0]

    def prelu(z, al):
        return jnp.where(z >= 0, z, al * z)

    t = jnp.dot(a, x, preferred_element_type=jnp.float32)
    h = (jnp.dot(t, w1r_ref[...], preferred_element_type=jnp.float32)
         + b1_ref[...]
         + jnp.dot(x, w1t_ref[...], preferred_element_type=jnp.float32))
    h = prelu(h, a1_ref[...])
    t = jnp.dot(a, h, preferred_element_type=jnp.float32)
    h = (jnp.dot(t, w2r_ref[...], preferred_element_type=jnp.float32)
         + b2_ref[...]
         + jnp.dot(h, w2t_ref[...], preferred_element_type=jnp.float32))
    h = prelu(h, a2_ref[...])
    t = jnp.dot(a, h, preferred_element_type=jnp.float32)
    h = (jnp.dot(t, w3r_ref[...], preferred_element_type=jnp.float32)
         + b3_ref[...]
         + jnp.dot(h, w3t_ref[...], preferred_element_type=jnp.float32))
    h = prelu(h, a3_ref[...])
    out_ref[0] = h


def _make_conn():
    full = lambda r, c: pl.BlockSpec((r, c), lambda g: (0, 0))
    return pl.pallas_call(
        _conn_body,
        grid=(G,),
        in_specs=[
            pl.BlockSpec((1, 87, 87), lambda g: (g, 0, 0)),
            pl.BlockSpec((1, 87, 10), lambda g: (g, 0, 0)),
            full(10, 20), full(1, 20), full(10, 20), full(1, 1),
            full(20, 20), full(1, 20), full(20, 20), full(1, 1),
            full(20, 5), full(1, 5), full(20, 5), full(1, 1),
        ],
        out_specs=pl.BlockSpec((1, 87, 5), lambda g: (g, 0, 0)),
        out_shape=jax.ShapeDtypeStruct((G, 87, 5), jnp.float32),
    )


def _final_body(p_ref, h_ref, dis_ref, fb_ref, g_ref, b_ref, batch_ref,
                conn_ref, w1a_ref, w1b_ref, hb1_ref, hg_ref, hbb_ref,
                w2_ref, hb2_ref, out_ref, acc_ref, seg_ref, cnt_ref):
    j = pl.program_id(0)
    p = p_ref[...]
    h = h_ref[...]
    dis = dis_ref[...]
    u = dis * p + h * (dis * dis) + fb_ref[...]
    u = jnp.maximum(u, 0.0)

    @pl.when(j == 0)
    def _():
        acc_ref[...] = jnp.zeros_like(acc_ref)
        seg_ref[...] = jnp.zeros_like(seg_ref)
        cnt_ref[...] = jnp.zeros_like(cnt_ref)

    acc_ref[0:1, 0:F] += jnp.sum(u, axis=0, keepdims=True)
    acc_ref[1:2, 0:F] += jnp.sum(u * u, axis=0, keepdims=True)

    b = batch_ref[0]
    mask = (lax.broadcasted_iota(jnp.int32, (G, BLK), 0) == b
            ).astype(jnp.float32)
    seg_ref[...] += jnp.dot(mask, u, preferred_element_type=jnp.float32)
    cnt_ref[...] += jnp.sum(mask, axis=1, keepdims=True)

    m = acc_ref[0:1, 0:F] / N
    v = acc_ref[1:2, 0:F] / N - m * m
    rstd = lax.rsqrt(v + 1e-5)
    cnt = jnp.maximum(cnt_ref[...], 1.0)
    mean_u = seg_ref[...] / cnt
    mesh_feat = (mean_u - m) * rstd * g_ref[...] + b_ref[...]
    z = (jnp.dot(mesh_feat, w1a_ref[...], preferred_element_type=jnp.float32)
         + jnp.dot(conn_ref[...], w1b_ref[...],
                   preferred_element_type=jnp.float32)
         + hb1_ref[...])
    z = jnp.maximum(z, 0.0)
    zm = jnp.mean(z, axis=0, keepdims=True)
    zv = jnp.mean(z * z, axis=0, keepdims=True) - zm * zm
    zn = (z - zm) * lax.rsqrt(zv + 1e-5) * hg_ref[...] + hbb_ref[...]
    out_ref[...] = (jnp.dot(zn, w2_ref[...], preferred_element_type=jnp.float32)
                    + hb2_ref[...])


def _make_final():
    full = lambda r, c: pl.BlockSpec((r, c), lambda j: (0, 0))
    return pl.pallas_call(
        _final_body,
        grid=(NB,),
        in_specs=[
            pl.BlockSpec((BLK, F), lambda j: (j, 0)),
            pl.BlockSpec((BLK, F), lambda j: (j, 0)),
            pl.BlockSpec((BLK, 1), lambda j: (j, 0)),
            full(1, F), full(1, F), full(1, F),
            pl.BlockSpec((1, 1, BLK), lambda j: (j, 0, 0)),
            full(G, 435), full(F, 10), full(435, 10), full(1, 10),
            full(1, 10), full(1, 10), full(10, 1), full(1, 1),
        ],
        out_specs=pl.BlockSpec((G, 1), lambda j: (0, 0)),
        out_shape=jax.ShapeDtypeStruct((G, 1), jnp.float32),
        scratch_shapes=[
            pltpu.VMEM((8, 128), jnp.float32),
            pltpu.VMEM((G, F), jnp.float32),
            pltpu.VMEM((G, 1), jnp.float32),
        ],
    )


_k1 = _make_k1()
_klayer = _make_layer()
_kconn = _make_conn()
_kfinal = _make_final()


def kernel(mesh_pos, mesh_norm, mesh_dha, mesh_x, mesh_edge_index, mesh_batch,
           conn_x, conn_adj,
           cw1_rel, cb1, cw1_root, ca1,
           cw2_rel, cb2, cw2_root, ca2,
           cw3_rel, cb3, cw3_root, ca3,
           bn0_g, bn0_b,
           fw1, fb1, bn1_g, bn1_b,
           fw2, fb2, bn2_g, bn2_b,
           fw3, fb3, bn3_g, bn3_b,
           fw4, fb4, bn4_g, bn4_b,
           hw1, hb1, hbn_g, hbn_b, hw2, hb2):
    r = lambda a: a.reshape(1, -1)
    x13 = jnp.concatenate([mesh_pos, mesh_norm, mesh_dha, mesh_x], axis=1)
    zeros_h = jnp.zeros((NPAD, FH), jnp.float32)
    ones_h = jnp.ones((N, FH), jnp.float32)

    degp = _make_agg()(ones_h, ones_h, mesh_edge_index, zeros_h)

    conn3 = _kconn(conn_adj, conn_x,
                   cw1_rel, r(cb1), cw1_root, r(ca1),
                   cw2_rel, r(cb2), cw2_root, r(ca2),
                   cw3_rel, r(cb3), cw3_root, r(ca3))
    conn_feat = conn3.reshape(G, 435)

    h, hs0, hs1, dis = _k1(x13, degp, r(bn0_g), r(bn0_b), fw1)

    fbs = (fb1, fb2, fb3)
    gs = (bn1_g, bn2_g, bn3_g)
    bs = (bn1_b, bn2_b, bn3_b)
    ws = (fw2, fw3, fw4)
    for i in range(3):
        p = _make_agg()(hs0, hs1, mesh_edge_index, zeros_h)
        h, hs0, hs1 = _klayer(p, h, dis, r(fbs[i]), r(gs[i]), r(bs[i]), ws[i])

    p = _make_agg()(hs0, hs1, mesh_edge_index, zeros_h)
    out = _kfinal(p, h, dis, r(fb4), r(bn4_g), r(bn4_b),
                  mesh_batch.reshape(NB, 1, BLK), conn_feat,
                  hw1[:F], hw1[F:], r(hb1), r(hbn_g), r(hbn_b), hw2, r(hb2))
    return out


# speedup vs baseline: 35.5930x; 1.0339x over previous
import functools

import jax
import jax.numpy as jnp
from jax import lax
from jax.experimental import pallas as pl
from jax.experimental.pallas import tpu as pltpu
from jax.experimental.pallas import tpu_sc as plsc

N = 50000
E = 1600000
G = 64
F = 32
NC, NS = 2, 16
NW = NC * NS
EPW = E // NW
CH = 2000
NPAD = 50048
ROWS_W = NPAD // NS
DEGW = 16

BLK = 5000
NB = N // BLK


FH = F // NC
EPS = E // NS
NCHS = EPS // CH


def _agg_body(hs0_hbm, hs1_hbm, ei_hbm, zeros_hbm, out_hbm,
              sidx0, sidx1, didx0, didx1, rows0, rows1, agg_sh, sem0, sem1):
    c = lax.axis_index("c")
    s = lax.axis_index("s")
    sidx = (sidx0, sidx1)
    didx = (didx0, didx1)
    rows = (rows0, rows1)
    sems = (sem0, sem1)

    def load_idx(i, b):
        base = s * EPS + i * CH
        pltpu.sync_copy(ei_hbm.at[0, pl.ds(base, CH)], sidx[b])
        pltpu.sync_copy(ei_hbm.at[1, pl.ds(base, CH)], didx[b])

    def start_gather(b):
        @pl.when(c == 0)
        def _():
            pltpu.async_copy(hs0_hbm.at[sidx[b]], rows[b], sems[b])

        @pl.when(c == 1)
        def _():
            pltpu.async_copy(hs1_hbm.at[sidx[b]], rows[b], sems[b])

    def wait_gather(b):
        pltpu.make_async_copy(hs0_hbm.at[sidx[b]], rows[b], sems[b]).wait()

    pltpu.sync_copy(zeros_hbm.at[pl.ds(s * ROWS_W, ROWS_W)],
                    agg_sh.at[pl.ds(s * ROWS_W, ROWS_W)])
    plsc.subcore_barrier()

    for b in range(2):
        load_idx(b, b)
        start_gather(b)

    def body(k, carry):
        for b in range(2):
            i = k * 2 + b
            wait_gather(b)
            pltpu.sync_copy(rows[b], agg_sh.at[didx[b]], add=True)

            @pl.when(i + 2 < NCHS)
            def _():
                load_idx(i + 2, b)
                start_gather(b)
        return carry

    lax.fori_loop(0, NCHS // 2, body, 0)
    plsc.subcore_barrier()
    pltpu.sync_copy(agg_sh.at[pl.ds(s * ROWS_W, ROWS_W)],
                    out_hbm.at[pl.ds(s * ROWS_W, ROWS_W), pl.ds(c * FH, FH)])


@functools.lru_cache(maxsize=None)
def _make_agg():
    return pl.kernel(
        _agg_body,
        out_type=jax.ShapeDtypeStruct((NPAD, F), jnp.float32),
        mesh=plsc.VectorSubcoreMesh(core_axis_name="c", subcore_axis_name="s"),
        scratch_types=[
            pltpu.VMEM((CH,), jnp.int32),
            pltpu.VMEM((CH,), jnp.int32),
            pltpu.VMEM((CH,), jnp.int32),
            pltpu.VMEM((CH,), jnp.int32),
            pltpu.VMEM((CH, FH), jnp.float32),
            pltpu.VMEM((CH, FH), jnp.float32),
            pltpu.VMEM_SHARED((NPAD, FH), jnp.float32),
            pltpu.SemaphoreType.DMA,
            pltpu.SemaphoreType.DMA,
        ],
        compiler_params=pltpu.CompilerParams(use_tc_tiling_on_sc=False),
    )


def _k1_body(x_ref, degp_ref, g0_ref, b0_ref, w_ref,
             h_ref, hs0_ref, hs1_ref, dis_ref, acc_ref, st_ref):
    ph = pl.program_id(0)
    j = pl.program_id(1)
    dp = degp_ref[...]
    dis = lax.rsqrt(1.0 + dp[:, 0:1])
    dis_ref[...] = dis

    @pl.when(ph == 0)
    def _():
        x = x_ref[...]

        @pl.when(j == 0)
        def _():
            acc_ref[...] = jnp.zeros_like(acc_ref)

        acc_ref[0:1, 0:13] += jnp.sum(x, axis=0, keepdims=True)
        acc_ref[1:2, 0:13] += jnp.sum(x * x, axis=0, keepdims=True)

    @pl.when(ph == 1)
    def _():
        @pl.when(j == 0)
        def _():
            m = acc_ref[0:1, 0:13] / N
            v = acc_ref[1:2, 0:13] / N - m * m
            scale = lax.rsqrt(v + 1e-5) * g0_ref[...]
            st_ref[0:1, 0:13] = scale
            st_ref[1:2, 0:13] = b0_ref[...] - m * scale

        xn = x_ref[...] * st_ref[0:1, 0:13] + st_ref[1:2, 0:13]
        h = jnp.dot(xn, w_ref[...], preferred_element_type=jnp.float32)
        h_ref[...] = h
        hsc = h * dis
        hs0_ref[...] = hsc[:, :FH]
        hs1_ref[...] = hsc[:, FH:]


def _make_k1():
    return pl.pallas_call(
        _k1_body,
        grid=(2, NB),
        in_specs=[
            pl.BlockSpec((BLK, 13), lambda ph, j: (j, 0)),
            pl.BlockSpec((BLK, F), lambda ph, j: (j, 0)),
            pl.BlockSpec((1, 13), lambda ph, j: (0, 0)),
            pl.BlockSpec((1, 13), lambda ph, j: (0, 0)),
            pl.BlockSpec((13, F), lambda ph, j: (0, 0)),
        ],
        out_specs=[
            pl.BlockSpec((BLK, F), lambda ph, j: (j, 0)),
            pl.BlockSpec((BLK, FH), lambda ph, j: (j, 0)),
            pl.BlockSpec((BLK, FH), lambda ph, j: (j, 0)),
            pl.BlockSpec((BLK, 1), lambda ph, j: (j, 0)),
        ],
        out_shape=[
            jax.ShapeDtypeStruct((N, F), jnp.float32),
            jax.ShapeDtypeStruct((N, FH), jnp.float32),
            jax.ShapeDtypeStruct((N, FH), jnp.float32),
            jax.ShapeDtypeStruct((N, 1), jnp.float32),
        ],
        scratch_shapes=[pltpu.VMEM((8, 128), jnp.float32),
                        pltpu.VMEM((8, 128), jnp.float32)],
    )


def _layer_body(p_ref, h_ref, dis_ref, fb_ref, g_ref, b_ref, w_ref,
                hn_ref, hs0_ref, hs1_ref, acc_ref, st_ref, u_ref):
    ph = pl.program_id(0)
    j = pl.program_id(1)
    dis = dis_ref[...]

    @pl.when(ph == 0)
    def _():
        u = dis * p_ref[...] + h_ref[...] * (dis * dis) + fb_ref[...]
        u = jnp.maximum(u, 0.0)
        u_ref[pl.ds(j * BLK, BLK), :] = u

        @pl.when(j == 0)
        def _():
            acc_ref[...] = jnp.zeros_like(acc_ref)

        acc_ref[0:1, 0:F] += jnp.sum(u, axis=0, keepdims=True)
        acc_ref[1:2, 0:F] += jnp.sum(u * u, axis=0, keepdims=True)

    @pl.when(ph == 1)
    def _():
        @pl.when(j == 0)
        def _():
            m = acc_ref[0:1, 0:F] / N
            v = acc_ref[1:2, 0:F] / N - m * m
            scale = lax.rsqrt(v + 1e-5) * g_ref[...]
            st_ref[0:1, 0:F] = scale
            st_ref[1:2, 0:F] = b_ref[...] - m * scale

        mx = u_ref[pl.ds(j * BLK, BLK), :] * st_ref[0:1, 0:F] + st_ref[1:2, 0:F]
        hn = jnp.dot(mx, w_ref[...], preferred_element_type=jnp.float32)
        hn_ref[...] = hn
        hsc = hn * dis
        hs0_ref[...] = hsc[:, :FH]
        hs1_ref[...] = hsc[:, FH:]


def _make_layer():
    return pl.pallas_call(
        _layer_body,
        grid=(2, NB),
        in_specs=[
            pl.BlockSpec((BLK, F), lambda ph, j: ((1 - ph) * j, 0)),
            pl.BlockSpec((BLK, F), lambda ph, j: ((1 - ph) * j, 0)),
            pl.BlockSpec((BLK, 1), lambda ph, j: (j, 0)),
            pl.BlockSpec((1, F), lambda ph, j: (0, 0)),
            pl.BlockSpec((1, F), lambda ph, j: (0, 0)),
            pl.BlockSpec((1, F), lambda ph, j: (0, 0)),
            pl.BlockSpec((F, F), lambda ph, j: (0, 0)),
        ],
        out_specs=[
            pl.BlockSpec((BLK, F), lambda ph, j: (j, 0)),
            pl.BlockSpec((BLK, FH), lambda ph, j: (j, 0)),
            pl.BlockSpec((BLK, FH), lambda ph, j: (j, 0)),
        ],
        out_shape=[
            jax.ShapeDtypeStruct((N, F), jnp.float32),
            jax.ShapeDtypeStruct((N, FH), jnp.float32),
            jax.ShapeDtypeStruct((N, FH), jnp.float32),
        ],
        scratch_shapes=[pltpu.VMEM((8, 128), jnp.float32),
                        pltpu.VMEM((8, 128), jnp.float32),
                        pltpu.VMEM((N, F), jnp.float32)],
    )


def _conn_body(adj_ref, cx_ref, w1r_ref, b1_ref, w1t_ref, a1_ref,
               w2r_ref, b2_ref, w2t_ref, a2_ref,
               w3r_ref, b3_ref, w3t_ref, a3_ref, out_ref):
    a = adj_ref[0]
    x = cx_ref[0]

    def prelu(z, al):
        return jnp.where(z >= 0, z, al * z)

    t = jnp.dot(a, x, preferred_element_type=jnp.float32)
    h = (jnp.dot(t, w1r_ref[...], preferred_element_type=jnp.float32)
         + b1_ref[...]
         + jnp.dot(x, w1t_ref[...], preferred_element_type=jnp.float32))
    h = prelu(h, a1_ref[...])
    t = jnp.dot(a, h, preferred_element_type=jnp.float32)
    h = (jnp.dot(t, w2r_ref[...], preferred_element_type=jnp.float32)
         + b2_ref[...]
         + jnp.dot(h, w2t_ref[...], preferred_element_type=jnp.float32))
    h = prelu(h, a2_ref[...])
    t = jnp.dot(a, h, preferred_element_type=jnp.float32)
    h = (jnp.dot(t, w3r_ref[...], preferred_element_type=jnp.float32)
         + b3_ref[...]
         + jnp.dot(h, w3t_ref[...], preferred_element_type=jnp.float32))
    h = prelu(h, a3_ref[...])
    out_ref[0] = h


def _make_conn():
    full = lambda r, c: pl.BlockSpec((r, c), lambda g: (0, 0))
    return pl.pallas_call(
        _conn_body,
        grid=(G,),
        in_specs=[
            pl.BlockSpec((1, 87, 87), lambda g: (g, 0, 0)),
            pl.BlockSpec((1, 87, 10), lambda g: (g, 0, 0)),
            full(10, 20), full(1, 20), full(10, 20), full(1, 1),
            full(20, 20), full(1, 20), full(20, 20), full(1, 1),
            full(20, 5), full(1, 5), full(20, 5), full(1, 1),
        ],
        out_specs=pl.BlockSpec((1, 87, 5), lambda g: (g, 0, 0)),
        out_shape=jax.ShapeDtypeStruct((G, 87, 5), jnp.float32),
    )


def _final_body(p_ref, h_ref, dis_ref, fb_ref, g_ref, b_ref, batch_ref,
                conn_ref, w1a_ref, w1b_ref, hb1_ref, hg_ref, hbb_ref,
                w2_ref, hb2_ref, out_ref, acc_ref, seg_ref, cnt_ref):
    j = pl.program_id(0)
    p = p_ref[...]
    h = h_ref[...]
    dis = dis_ref[...]
    u = dis * p + h * (dis * dis) + fb_ref[...]
    u = jnp.maximum(u, 0.0)

    @pl.when(j == 0)
    def _():
        acc_ref[...] = jnp.zeros_like(acc_ref)
        seg_ref[...] = jnp.zeros_like(seg_ref)
        cnt_ref[...] = jnp.zeros_like(cnt_ref)

    acc_ref[0:1, 0:F] += jnp.sum(u, axis=0, keepdims=True)
    acc_ref[1:2, 0:F] += jnp.sum(u * u, axis=0, keepdims=True)

    b = batch_ref[0]
    mask = (lax.broadcasted_iota(jnp.int32, (G, BLK), 0) == b
            ).astype(jnp.float32)
    seg_ref[...] += jnp.dot(mask, u, preferred_element_type=jnp.float32)
    cnt_ref[...] += jnp.sum(mask, axis=1, keepdims=True)

    @pl.when(j == NB - 1)
    def _():
        m = acc_ref[0:1, 0:F] / N
        v = acc_ref[1:2, 0:F] / N - m * m
        rstd = lax.rsqrt(v + 1e-5)
        cnt = jnp.maximum(cnt_ref[...], 1.0)
        mean_u = seg_ref[...] / cnt
        mesh_feat = (mean_u - m) * rstd * g_ref[...] + b_ref[...]
        z = (jnp.dot(mesh_feat, w1a_ref[...],
                     preferred_element_type=jnp.float32)
             + jnp.dot(conn_ref[...], w1b_ref[...],
                       preferred_element_type=jnp.float32)
             + hb1_ref[...])
        z = jnp.maximum(z, 0.0)
        zm = jnp.mean(z, axis=0, keepdims=True)
        zv = jnp.mean(z * z, axis=0, keepdims=True) - zm * zm
        zn = (z - zm) * lax.rsqrt(zv + 1e-5) * hg_ref[...] + hbb_ref[...]
        out_ref[...] = (jnp.dot(zn, w2_ref[...],
                                preferred_element_type=jnp.float32)
                        + hb2_ref[...])


def _make_final():
    full = lambda r, c: pl.BlockSpec((r, c), lambda j: (0, 0))
    return pl.pallas_call(
        _final_body,
        grid=(NB,),
        in_specs=[
            pl.BlockSpec((BLK, F), lambda j: (j, 0)),
            pl.BlockSpec((BLK, F), lambda j: (j, 0)),
            pl.BlockSpec((BLK, 1), lambda j: (j, 0)),
            full(1, F), full(1, F), full(1, F),
            pl.BlockSpec((1, 1, BLK), lambda j: (j, 0, 0)),
            full(G, 435), full(F, 10), full(435, 10), full(1, 10),
            full(1, 10), full(1, 10), full(10, 1), full(1, 1),
        ],
        out_specs=pl.BlockSpec((G, 1), lambda j: (0, 0)),
        out_shape=jax.ShapeDtypeStruct((G, 1), jnp.float32),
        scratch_shapes=[
            pltpu.VMEM((8, 128), jnp.float32),
            pltpu.VMEM((G, F), jnp.float32),
            pltpu.VMEM((G, 1), jnp.float32),
        ],
    )


_k1 = _make_k1()
_klayer = _make_layer()
_kconn = _make_conn()
_kfinal = _make_final()


def kernel(mesh_pos, mesh_norm, mesh_dha, mesh_x, mesh_edge_index, mesh_batch,
           conn_x, conn_adj,
           cw1_rel, cb1, cw1_root, ca1,
           cw2_rel, cb2, cw2_root, ca2,
           cw3_rel, cb3, cw3_root, ca3,
           bn0_g, bn0_b,
           fw1, fb1, bn1_g, bn1_b,
           fw2, fb2, bn2_g, bn2_b,
           fw3, fb3, bn3_g, bn3_b,
           fw4, fb4, bn4_g, bn4_b,
           hw1, hb1, hbn_g, hbn_b, hw2, hb2):
    r = lambda a: a.reshape(1, -1)
    x13 = jnp.concatenate([mesh_pos, mesh_norm, mesh_dha, mesh_x], axis=1)
    zeros_h = jnp.zeros((NPAD, FH), jnp.float32)
    ones_h = jnp.ones((N, FH), jnp.float32)

    degp = _make_agg()(ones_h, ones_h, mesh_edge_index, zeros_h)

    conn3 = _kconn(conn_adj, conn_x,
                   cw1_rel, r(cb1), cw1_root, r(ca1),
                   cw2_rel, r(cb2), cw2_root, r(ca2),
                   cw3_rel, r(cb3), cw3_root, r(ca3))
    conn_feat = conn3.reshape(G, 435)

    h, hs0, hs1, dis = _k1(x13, degp, r(bn0_g), r(bn0_b), fw1)

    fbs = (fb1, fb2, fb3)
    gs = (bn1_g, bn2_g, bn3_g)
    bs = (bn1_b, bn2_b, bn3_b)
    ws = (fw2, fw3, fw4)
    for i in range(3):
        p = _make_agg()(hs0, hs1, mesh_edge_index, zeros_h)
        h, hs0, hs1 = _klayer(p, h, dis, r(fbs[i]), r(gs[i]), r(bs[i]), ws[i])

    p = _make_agg()(hs0, hs1, mesh_edge_index, zeros_h)
    out = _kfinal(p, h, dis, r(fb4), r(bn4_g), r(bn4_b),
                  mesh_batch.reshape(NB, 1, BLK), conn_feat,
                  hw1[:F], hw1[F:], r(hb1), r(hbn_g), r(hbn_b), hw2, r(hb2))
    return out


# speedup vs baseline: 37.9129x; 1.0652x over previous
import functools

import jax
import jax.numpy as jnp
from jax import lax
from jax.experimental import pallas as pl
from jax.experimental.pallas import tpu as pltpu
from jax.experimental.pallas import tpu_sc as plsc

N = 50000
E = 1600000
G = 64
F = 32
NC, NS = 2, 16
NW = NC * NS
EPW = E // NW
CH = 1000
NPAD = 50048
ROWS_W = NPAD // NS
DEGW = 16

BLK = 5000
NB = N // BLK


FH = F // NC
EPS = E // NS
NCHS = EPS // CH


NBUF = 4


def _agg_body(hs0_hbm, hs1_hbm, ei_hbm, zeros_hbm, ones_hbm, mode_hbm,
              out_hbm, sidx0, sidx1, sidx2, sidx3, didx0, didx1, didx2, didx3,
              rows0, rows1, rows2, rows3, agg_sh, mode_sm,
              sg0, sg1, sg2, sg3, ss0, ss1, ss2, ss3):
    c = lax.axis_index("c")
    s = lax.axis_index("s")
    sidx = (sidx0, sidx1, sidx2, sidx3)
    didx = (didx0, didx1, didx2, didx3)
    rows = (rows0, rows1, rows2, rows3)
    semg = (sg0, sg1, sg2, sg3)
    sems = (ss0, ss1, ss2, ss3)

    pltpu.sync_copy(mode_hbm, mode_sm)
    do_gather = jnp.max(mode_sm[...]) == 1

    def load_idx(i, b):
        base = s * EPS + i * CH

        @pl.when(do_gather)
        def _():
            pltpu.sync_copy(ei_hbm.at[0, pl.ds(base, CH)], sidx[b])

        pltpu.sync_copy(ei_hbm.at[1, pl.ds(base, CH)], didx[b])

    def start_gather(b):
        @pl.when(jnp.logical_and(do_gather, c == 0))
        def _():
            pltpu.async_copy(hs0_hbm.at[sidx[b]], rows[b], semg[b])

        @pl.when(jnp.logical_and(do_gather, c == 1))
        def _():
            pltpu.async_copy(hs1_hbm.at[sidx[b]], rows[b], semg[b])

    def wait_gather(b):
        @pl.when(do_gather)
        def _():
            pltpu.make_async_copy(hs0_hbm.at[sidx[b]], rows[b],
                                  semg[b]).wait()

    def start_scatter(b):
        pltpu.async_copy(rows[b], agg_sh.at[didx[b]], sems[b], add=True)

    def wait_scatter(b):
        pltpu.make_async_copy(rows[b], agg_sh.at[didx[b]], sems[b]).wait()

    pltpu.sync_copy(zeros_hbm.at[pl.ds(s * ROWS_W, ROWS_W)],
                    agg_sh.at[pl.ds(s * ROWS_W, ROWS_W)])

    @pl.when(jnp.logical_not(do_gather))
    def _():
        for b in range(NBUF):
            pltpu.sync_copy(ones_hbm, rows[b])

    plsc.subcore_barrier()

    for j in range(2):
        load_idx(j, j)
        start_gather(j)

    def body(k, carry):
        for b in range(NBUF):
            i = k * NBUF + b
            j = i + 2
            bj = (b + 2) % NBUF

            @pl.when(j < NCHS)
            def _():
                @pl.when(j >= NBUF)
                def _():
                    wait_scatter(bj)

                load_idx(j, bj)
                start_gather(bj)

            wait_gather(b)
            start_scatter(b)
        return carry

    lax.fori_loop(0, NCHS // NBUF, body, 0)
    for b in range(NBUF):
        wait_scatter(b)
    plsc.subcore_barrier()
    pltpu.sync_copy(agg_sh.at[pl.ds(s * ROWS_W, ROWS_W)],
                    out_hbm.at[pl.ds(s * ROWS_W, ROWS_W), pl.ds(c * FH, FH)])


@functools.lru_cache(maxsize=None)
def _make_agg():
    return pl.kernel(
        _agg_body,
        out_type=jax.ShapeDtypeStruct((NPAD, F), jnp.float32),
        mesh=plsc.VectorSubcoreMesh(core_axis_name="c", subcore_axis_name="s"),
        scratch_types=(
            [pltpu.VMEM((CH,), jnp.int32)] * 8
            + [pltpu.VMEM((CH, FH), jnp.float32)] * 4
            + [pltpu.VMEM_SHARED((NPAD, FH), jnp.float32),
               pltpu.VMEM((16,), jnp.int32)]
            + [pltpu.SemaphoreType.DMA] * 8
        ),
        compiler_params=pltpu.CompilerParams(use_tc_tiling_on_sc=False,
                                             needs_layout_passes=False),
    )


def _k1_body(x_ref, degp_ref, g0_ref, b0_ref, w_ref,
             h_ref, hs0_ref, hs1_ref, dis_ref, acc_ref, st_ref):
    ph = pl.program_id(0)
    j = pl.program_id(1)
    dp = degp_ref[...]
    dis = lax.rsqrt(1.0 + dp[:, 0:1])
    dis_ref[...] = dis

    @pl.when(ph == 0)
    def _():
        x = x_ref[...]

        @pl.when(j == 0)
        def _():
            acc_ref[...] = jnp.zeros_like(acc_ref)

        acc_ref[0:1, 0:13] += jnp.sum(x, axis=0, keepdims=True)
        acc_ref[1:2, 0:13] += jnp.sum(x * x, axis=0, keepdims=True)

    @pl.when(ph == 1)
    def _():
        @pl.when(j == 0)
        def _():
            m = acc_ref[0:1, 0:13] / N
            v = acc_ref[1:2, 0:13] / N - m * m
            scale = lax.rsqrt(v + 1e-5) * g0_ref[...]
            st_ref[0:1, 0:13] = scale
            st_ref[1:2, 0:13] = b0_ref[...] - m * scale

        xn = x_ref[...] * st_ref[0:1, 0:13] + st_ref[1:2, 0:13]
        h = jnp.dot(xn, w_ref[...], preferred_element_type=jnp.float32)
        h_ref[...] = h
        hsc = h * dis
        hs0_ref[...] = hsc[:, :FH]
        hs1_ref[...] = hsc[:, FH:]


def _make_k1():
    return pl.pallas_call(
        _k1_body,
        grid=(2, NB),
        in_specs=[
            pl.BlockSpec((BLK, 13), lambda ph, j: (j, 0)),
            pl.BlockSpec((BLK, F), lambda ph, j: (j, 0)),
            pl.BlockSpec((1, 13), lambda ph, j: (0, 0)),
            pl.BlockSpec((1, 13), lambda ph, j: (0, 0)),
            pl.BlockSpec((13, F), lambda ph, j: (0, 0)),
        ],
        out_specs=[
            pl.BlockSpec((BLK, F), lambda ph, j: (j, 0)),
            pl.BlockSpec((BLK, FH), lambda ph, j: (j, 0)),
            pl.BlockSpec((BLK, FH), lambda ph, j: (j, 0)),
            pl.BlockSpec((BLK, 1), lambda ph, j: (j, 0)),
        ],
        out_shape=[
            jax.ShapeDtypeStruct((N, F), jnp.float32),
            jax.ShapeDtypeStruct((N, FH), jnp.float32),
            jax.ShapeDtypeStruct((N, FH), jnp.float32),
            jax.ShapeDtypeStruct((N, 1), jnp.float32),
        ],
        scratch_shapes=[pltpu.VMEM((8, 128), jnp.float32),
                        pltpu.VMEM((8, 128), jnp.float32)],
    )


def _layer_body(p_ref, h_ref, dis_ref, fb_ref, g_ref, b_ref, w_ref,
                hn_ref, hs0_ref, hs1_ref, acc_ref, st_ref, u_ref):
    ph = pl.program_id(0)
    j = pl.program_id(1)
    dis = dis_ref[...]

    @pl.when(ph == 0)
    def _():
        u = dis * p_ref[...] + h_ref[...] * (dis * dis) + fb_ref[...]
        u = jnp.maximum(u, 0.0)
        u_ref[pl.ds(j * BLK, BLK), :] = u

        @pl.when(j == 0)
        def _():
            acc_ref[...] = jnp.zeros_like(acc_ref)

        acc_ref[0:1, 0:F] += jnp.sum(u, axis=0, keepdims=True)
        acc_ref[1:2, 0:F] += jnp.sum(u * u, axis=0, keepdims=True)

    @pl.when(ph == 1)
    def _():
        @pl.when(j == 0)
        def _():
            m = acc_ref[0:1, 0:F] / N
            v = acc_ref[1:2, 0:F] / N - m * m
            scale = lax.rsqrt(v + 1e-5) * g_ref[...]
            st_ref[0:1, 0:F] = scale
            st_ref[1:2, 0:F] = b_ref[...] - m * scale

        mx = u_ref[pl.ds(j * BLK, BLK), :] * st_ref[0:1, 0:F] + st_ref[1:2, 0:F]
        hn = jnp.dot(mx, w_ref[...], preferred_element_type=jnp.float32)
        hn_ref[...] = hn
        hsc = hn * dis
        hs0_ref[...] = hsc[:, :FH]
        hs1_ref[...] = hsc[:, FH:]


def _make_layer():
    return pl.pallas_call(
        _layer_body,
        grid=(2, NB),
        in_specs=[
            pl.BlockSpec((BLK, F), lambda ph, j: ((1 - ph) * j, 0)),
            pl.BlockSpec((BLK, F), lambda ph, j: ((1 - ph) * j, 0)),
            pl.BlockSpec((BLK, 1), lambda ph, j: (j, 0)),
            pl.BlockSpec((1, F), lambda ph, j: (0, 0)),
            pl.BlockSpec((1, F), lambda ph, j: (0, 0)),
            pl.BlockSpec((1, F), lambda ph, j: (0, 0)),
            pl.BlockSpec((F, F), lambda ph, j: (0, 0)),
        ],
        out_specs=[
            pl.BlockSpec((BLK, F), lambda ph, j: (j, 0)),
            pl.BlockSpec((BLK, FH), lambda ph, j: (j, 0)),
            pl.BlockSpec((BLK, FH), lambda ph, j: (j, 0)),
        ],
        out_shape=[
            jax.ShapeDtypeStruct((N, F), jnp.float32),
            jax.ShapeDtypeStruct((N, FH), jnp.float32),
            jax.ShapeDtypeStruct((N, FH), jnp.float32),
        ],
        scratch_shapes=[pltpu.VMEM((8, 128), jnp.float32),
                        pltpu.VMEM((8, 128), jnp.float32),
                        pltpu.VMEM((N, F), jnp.float32)],
    )


def _conn_body(adj_ref, cx_ref, w1r_ref, b1_ref, w1t_ref, a1_ref,
               w2r_ref, b2_ref, w2t_ref, a2_ref,
               w3r_ref, b3_ref, w3t_ref, a3_ref, out_ref):
    a = adj_ref[0]
    x = cx_ref[0]

    def prelu(z, al):
        return jnp.where(z >= 0, z, al * z)

    t = jnp.dot(a, x, preferred_element_type=jnp.float32)
    h = (jnp.dot(t, w1r_ref[...], preferred_element_type=jnp.float32)
         + b1_ref[...]
         + jnp.dot(x, w1t_ref[...], preferred_element_type=jnp.float32))
    h = prelu(h, a1_ref[...])
    t = jnp.dot(a, h, preferred_element_type=jnp.float32)
    h = (jnp.dot(t, w2r_ref[...], preferred_element_type=jnp.float32)
         + b2_ref[...]
         + jnp.dot(h, w2t_ref[...], preferred_element_type=jnp.float32))
    h = prelu(h, a2_ref[...])
    t = jnp.dot(a, h, preferred_element_type=jnp.float32)
    h = (jnp.dot(t, w3r_ref[...], preferred_element_type=jnp.float32)
         + b3_ref[...]
         + jnp.dot(h, w3t_ref[...], preferred_element_type=jnp.float32))
    h = prelu(h, a3_ref[...])
    out_ref[0] = h


def _make_conn():
    full = lambda r, c: pl.BlockSpec((r, c), lambda g: (0, 0))
    return pl.pallas_call(
        _conn_body,
        grid=(G,),
        in_specs=[
            pl.BlockSpec((1, 87, 87), lambda g: (g, 0, 0)),
            pl.BlockSpec((1, 87, 10), lambda g: (g, 0, 0)),
            full(10, 20), full(1, 20), full(10, 20), full(1, 1),
            full(20, 20), full(1, 20), full(20, 20), full(1, 1),
            full(20, 5), full(1, 5), full(20, 5), full(1, 1),
        ],
        out_specs=pl.BlockSpec((1, 87, 5), lambda g: (g, 0, 0)),
        out_shape=jax.ShapeDtypeStruct((G, 87, 5), jnp.float32),
    )


def _final_body(p_ref, h_ref, dis_ref, fb_ref, g_ref, b_ref, batch_ref,
                conn_ref, w1a_ref, w1b_ref, hb1_ref, hg_ref, hbb_ref,
                w2_ref, hb2_ref, out_ref, acc_ref, seg_ref, cnt_ref):
    j = pl.program_id(0)
    p = p_ref[...]
    h = h_ref[...]
    dis = dis_ref[...]
    u = dis * p + h * (dis * dis) + fb_ref[...]
    u = jnp.maximum(u, 0.0)

    @pl.when(j == 0)
    def _():
        acc_ref[...] = jnp.zeros_like(acc_ref)
        seg_ref[...] = jnp.zeros_like(seg_ref)
        cnt_ref[...] = jnp.zeros_like(cnt_ref)

    acc_ref[0:1, 0:F] += jnp.sum(u, axis=0, keepdims=True)
    acc_ref[1:2, 0:F] += jnp.sum(u * u, axis=0, keepdims=True)

    b = batch_ref[0]
    mask = (lax.broadcasted_iota(jnp.int32, (G, BLK), 0) == b
            ).astype(jnp.float32)
    seg_ref[...] += jnp.dot(mask, u, preferred_element_type=jnp.float32)
    cnt_ref[...] += jnp.sum(mask, axis=1, keepdims=True)

    @pl.when(j == NB - 1)
    def _():
        m = acc_ref[0:1, 0:F] / N
        v = acc_ref[1:2, 0:F] / N - m * m
        rstd = lax.rsqrt(v + 1e-5)
        cnt = jnp.maximum(cnt_ref[...], 1.0)
        mean_u = seg_ref[...] / cnt
        mesh_feat = (mean_u - m) * rstd * g_ref[...] + b_ref[...]
        z = (jnp.dot(mesh_feat, w1a_ref[...],
                     preferred_element_type=jnp.float32)
             + jnp.dot(conn_ref[...], w1b_ref[...],
                       preferred_element_type=jnp.float32)
             + hb1_ref[...])
        z = jnp.maximum(z, 0.0)
        zm = jnp.mean(z, axis=0, keepdims=True)
        zv = jnp.mean(z * z, axis=0, keepdims=True) - zm * zm
        zn = (z - zm) * lax.rsqrt(zv + 1e-5) * hg_ref[...] + hbb_ref[...]
        out_ref[...] = (jnp.dot(zn, w2_ref[...],
                                preferred_element_type=jnp.float32)
                        + hb2_ref[...])


def _make_final():
    full = lambda r, c: pl.BlockSpec((r, c), lambda j: (0, 0))
    return pl.pallas_call(
        _final_body,
        grid=(NB,),
        in_specs=[
            pl.BlockSpec((BLK, F), lambda j: (j, 0)),
            pl.BlockSpec((BLK, F), lambda j: (j, 0)),
            pl.BlockSpec((BLK, 1), lambda j: (j, 0)),
            full(1, F), full(1, F), full(1, F),
            pl.BlockSpec((1, 1, BLK), lambda j: (j, 0, 0)),
            full(G, 435), full(F, 10), full(435, 10), full(1, 10),
            full(1, 10), full(1, 10), full(10, 1), full(1, 1),
        ],
        out_specs=pl.BlockSpec((G, 1), lambda j: (0, 0)),
        out_shape=jax.ShapeDtypeStruct((G, 1), jnp.float32),
        scratch_shapes=[
            pltpu.VMEM((8, 128), jnp.float32),
            pltpu.VMEM((G, F), jnp.float32),
            pltpu.VMEM((G, 1), jnp.float32),
        ],
    )


_k1 = _make_k1()
_klayer = _make_layer()
_kconn = _make_conn()
_kfinal = _make_final()


def kernel(mesh_pos, mesh_norm, mesh_dha, mesh_x, mesh_edge_index, mesh_batch,
           conn_x, conn_adj,
           cw1_rel, cb1, cw1_root, ca1,
           cw2_rel, cb2, cw2_root, ca2,
           cw3_rel, cb3, cw3_root, ca3,
           bn0_g, bn0_b,
           fw1, fb1, bn1_g, bn1_b,
           fw2, fb2, bn2_g, bn2_b,
           fw3, fb3, bn3_g, bn3_b,
           fw4, fb4, bn4_g, bn4_b,
           hw1, hb1, hbn_g, hbn_b, hw2, hb2):
    r = lambda a: a.reshape(1, -1)
    x13 = jnp.concatenate([mesh_pos, mesh_norm, mesh_dha, mesh_x], axis=1)
    zeros_h = jnp.zeros((NPAD, FH), jnp.float32)
    ones_h = jnp.ones((N, FH), jnp.float32)
    ones_c = jnp.ones((CH, FH), jnp.float32)
    mode0 = jnp.zeros((16,), jnp.int32)
    mode1 = jnp.ones((16,), jnp.int32)

    degp = _make_agg()(ones_h, ones_h, mesh_edge_index, zeros_h, ones_c,
                       mode0)

    conn3 = _kconn(conn_adj, conn_x,
                   cw1_rel, r(cb1), cw1_root, r(ca1),
                   cw2_rel, r(cb2), cw2_root, r(ca2),
                   cw3_rel, r(cb3), cw3_root, r(ca3))
    conn_feat = conn3.reshape(G, 435)

    h, hs0, hs1, dis = _k1(x13, degp, r(bn0_g), r(bn0_b), fw1)

    fbs = (fb1, fb2, fb3)
    gs = (bn1_g, bn2_g, bn3_g)
    bs = (bn1_b, bn2_b, bn3_b)
    ws = (fw2, fw3, fw4)
    for i in range(3):
        p = _make_agg()(hs0, hs1, mesh_edge_index, zeros_h, ones_c, mode1)
        h, hs0, hs1 = _klayer(p, h, dis, r(fbs[i]), r(gs[i]), r(bs[i]), ws[i])

    p = _make_agg()(hs0, hs1, mesh_edge_index, zeros_h, ones_c, mode1)
    out = _kfinal(p, h, dis, r(fb4), r(bn4_g), r(bn4_b),
                  mesh_batch.reshape(NB, 1, BLK), conn_feat,
                  hw1[:F], hw1[F:], r(hb1), r(hbn_g), r(hbn_b), hw2, r(hb2))
    return out


# speedup vs baseline: 39.1456x; 1.0325x over previous
import functools

import jax
import jax.numpy as jnp
from jax import lax
from jax.experimental import pallas as pl
from jax.experimental.pallas import tpu as pltpu
from jax.experimental.pallas import tpu_sc as plsc

N = 50000
E = 1600000
G = 64
F = 32
NC, NS = 2, 16
NW = NC * NS
EPW = E // NW
CH = 1000
NPAD = 50048
ROWS_W = NPAD // NS
DEGW = 16

BLK = 5000
NB = N // BLK


FH = F // NC
EPS = E // NS
NCHS = EPS // CH


NBUF = 4


def _agg_body(hs0_hbm, hs1_hbm, ei_hbm, zeros_hbm, ones_hbm, mode_hbm,
              out_hbm, sidx0, sidx1, sidx2, sidx3, didx0, didx1, didx2, didx3,
              rows0, rows1, rows2, rows3, agg_sh, mode_sm,
              sg0, sg1, sg2, sg3, ss0, ss1, ss2, ss3):
    c = lax.axis_index("c")
    s = lax.axis_index("s")
    sidx = (sidx0, sidx1, sidx2, sidx3)
    didx = (didx0, didx1, didx2, didx3)
    rows = (rows0, rows1, rows2, rows3)
    semg = (sg0, sg1, sg2, sg3)
    sems = (ss0, ss1, ss2, ss3)

    pltpu.sync_copy(mode_hbm, mode_sm)
    do_gather = jnp.max(mode_sm[...]) == 1

    def load_idx(i, b):
        base = s * EPS + i * CH

        @pl.when(do_gather)
        def _():
            pltpu.sync_copy(ei_hbm.at[0, pl.ds(base, CH)], sidx[b])

        pltpu.sync_copy(ei_hbm.at[1, pl.ds(base, CH)], didx[b])

    def start_gather(b):
        @pl.when(jnp.logical_and(do_gather, c == 0))
        def _():
            pltpu.async_copy(hs0_hbm.at[sidx[b]], rows[b], semg[b])

        @pl.when(jnp.logical_and(do_gather, c == 1))
        def _():
            pltpu.async_copy(hs1_hbm.at[sidx[b]], rows[b], semg[b])

    def wait_gather(b):
        @pl.when(do_gather)
        def _():
            pltpu.make_async_copy(hs0_hbm.at[sidx[b]], rows[b],
                                  semg[b]).wait()

    def start_scatter(b):
        pltpu.async_copy(rows[b], agg_sh.at[didx[b]], sems[b], add=True)

    def wait_scatter(b):
        pltpu.make_async_copy(rows[b], agg_sh.at[didx[b]], sems[b]).wait()

    pltpu.sync_copy(zeros_hbm.at[pl.ds(s * ROWS_W, ROWS_W)],
                    agg_sh.at[pl.ds(s * ROWS_W, ROWS_W)])

    @pl.when(jnp.logical_not(do_gather))
    def _():
        for b in range(NBUF):
            pltpu.sync_copy(ones_hbm, rows[b])

    plsc.subcore_barrier()

    for j in range(2):
        load_idx(j, j)
        start_gather(j)

    def body(k, carry):
        for b in range(NBUF):
            i = k * NBUF + b
            j = i + 2
            bj = (b + 2) % NBUF

            @pl.when(j < NCHS)
            def _():
                @pl.when(j >= NBUF)
                def _():
                    wait_scatter(bj)

                load_idx(j, bj)
                start_gather(bj)

            wait_gather(b)
            start_scatter(b)
        return carry

    lax.fori_loop(0, NCHS // NBUF, body, 0)
    for b in range(NBUF):
        wait_scatter(b)
    plsc.subcore_barrier()
    pltpu.sync_copy(agg_sh.at[pl.ds(s * ROWS_W, ROWS_W)],
                    out_hbm.at[pl.ds(s * ROWS_W, ROWS_W), pl.ds(c * FH, FH)])


@functools.lru_cache(maxsize=None)
def _make_agg():
    return pl.kernel(
        _agg_body,
        out_type=jax.ShapeDtypeStruct((NPAD, F), jnp.float32),
        mesh=plsc.VectorSubcoreMesh(core_axis_name="c", subcore_axis_name="s"),
        scratch_types=(
            [pltpu.VMEM((CH,), jnp.int32)] * 8
            + [pltpu.VMEM((CH, FH), jnp.float32)] * 4
            + [pltpu.VMEM_SHARED((NPAD, FH), jnp.float32),
               pltpu.VMEM((16,), jnp.int32)]
            + [pltpu.SemaphoreType.DMA] * 8
        ),
        compiler_params=pltpu.CompilerParams(use_tc_tiling_on_sc=False,
                                             needs_layout_passes=False),
    )


def _k1a_body(x_ref, sums_ref, acc_ref):
    j = pl.program_id(0)
    x = x_ref[...]

    @pl.when(j == 0)
    def _():
        acc_ref[...] = jnp.zeros_like(acc_ref)

    acc_ref[0:1, 0:13] += jnp.sum(x, axis=0, keepdims=True)
    acc_ref[1:2, 0:13] += jnp.sum(x * x, axis=0, keepdims=True)

    @pl.when(j == NB - 1)
    def _():
        sums_ref[...] = acc_ref[...]


def _make_k1a():
    return pl.pallas_call(
        _k1a_body,
        grid=(NB,),
        in_specs=[pl.BlockSpec((BLK, 13), lambda j: (j, 0))],
        out_specs=pl.BlockSpec((8, 128), lambda j: (0, 0)),
        out_shape=jax.ShapeDtypeStruct((8, 128), jnp.float32),
        scratch_shapes=[pltpu.VMEM((8, 128), jnp.float32)],
    )


def _k1_body(x_ref, sums_ref, degp_ref, g0_ref, b0_ref, w_ref,
             h_ref, hs0_ref, hs1_ref, dis_ref, st_ref):
    j = pl.program_id(0)
    dp = degp_ref[...]
    dis = lax.rsqrt(1.0 + dp[:, 0:1])
    dis_ref[...] = dis

    @pl.when(j == 0)
    def _():
        m = sums_ref[0:1, 0:13] / N
        v = sums_ref[1:2, 0:13] / N - m * m
        scale = lax.rsqrt(v + 1e-5) * g0_ref[...]
        st_ref[0:1, 0:13] = scale
        st_ref[1:2, 0:13] = b0_ref[...] - m * scale

    xn = x_ref[...] * st_ref[0:1, 0:13] + st_ref[1:2, 0:13]
    h = jnp.dot(xn, w_ref[...], preferred_element_type=jnp.float32)
    h_ref[...] = h
    hsc = h * dis
    hs0_ref[...] = hsc[:, :FH]
    hs1_ref[...] = hsc[:, FH:]


def _make_k1():
    return pl.pallas_call(
        _k1_body,
        grid=(NB,),
        in_specs=[
            pl.BlockSpec((BLK, 13), lambda j: (j, 0)),
            pl.BlockSpec((8, 128), lambda j: (0, 0)),
            pl.BlockSpec((BLK, F), lambda j: (j, 0)),
            pl.BlockSpec((1, 13), lambda j: (0, 0)),
            pl.BlockSpec((1, 13), lambda j: (0, 0)),
            pl.BlockSpec((13, F), lambda j: (0, 0)),
        ],
        out_specs=[
            pl.BlockSpec((BLK, F), lambda j: (j, 0)),
            pl.BlockSpec((BLK, FH), lambda j: (j, 0)),
            pl.BlockSpec((BLK, FH), lambda j: (j, 0)),
            pl.BlockSpec((BLK, 1), lambda j: (j, 0)),
        ],
        out_shape=[
            jax.ShapeDtypeStruct((N, F), jnp.float32),
            jax.ShapeDtypeStruct((N, FH), jnp.float32),
            jax.ShapeDtypeStruct((N, FH), jnp.float32),
            jax.ShapeDtypeStruct((N, 1), jnp.float32),
        ],
        scratch_shapes=[pltpu.VMEM((8, 128), jnp.float32)],
    )


def _layer_body(p_ref, h_ref, dis_ref, fb_ref, g_ref, b_ref, w_ref,
                hn_ref, hs0_ref, hs1_ref, acc_ref, st_ref, u_ref):
    ph = pl.program_id(0)
    j = pl.program_id(1)
    dis = dis_ref[...]

    @pl.when(ph == 0)
    def _():
        u = dis * p_ref[...] + h_ref[...] * (dis * dis) + fb_ref[...]
        u = jnp.maximum(u, 0.0)
        u_ref[pl.ds(j * BLK, BLK), :] = u

        @pl.when(j == 0)
        def _():
            acc_ref[...] = jnp.zeros_like(acc_ref)

        acc_ref[0:1, 0:F] += jnp.sum(u, axis=0, keepdims=True)
        acc_ref[1:2, 0:F] += jnp.sum(u * u, axis=0, keepdims=True)

    @pl.when(ph == 1)
    def _():
        @pl.when(j == 0)
        def _():
            m = acc_ref[0:1, 0:F] / N
            v = acc_ref[1:2, 0:F] / N - m * m
            scale = lax.rsqrt(v + 1e-5) * g_ref[...]
            st_ref[0:1, 0:F] = scale
            st_ref[1:2, 0:F] = b_ref[...] - m * scale

        mx = u_ref[pl.ds(j * BLK, BLK), :] * st_ref[0:1, 0:F] + st_ref[1:2, 0:F]
        hn = jnp.dot(mx, w_ref[...], preferred_element_type=jnp.float32)
        hn_ref[...] = hn
        hsc = hn * dis
        hs0_ref[...] = hsc[:, :FH]
        hs1_ref[...] = hsc[:, FH:]


def _make_layer():
    return pl.pallas_call(
        _layer_body,
        grid=(2, NB),
        in_specs=[
            pl.BlockSpec((BLK, F), lambda ph, j: ((1 - ph) * j, 0)),
            pl.BlockSpec((BLK, F), lambda ph, j: ((1 - ph) * j, 0)),
            pl.BlockSpec((BLK, 1), lambda ph, j: (j, 0)),
            pl.BlockSpec((1, F), lambda ph, j: (0, 0)),
            pl.BlockSpec((1, F), lambda ph, j: (0, 0)),
            pl.BlockSpec((1, F), lambda ph, j: (0, 0)),
            pl.BlockSpec((F, F), lambda ph, j: (0, 0)),
        ],
        out_specs=[
            pl.BlockSpec((BLK, F), lambda ph, j: (j, 0)),
            pl.BlockSpec((BLK, FH), lambda ph, j: (j, 0)),
            pl.BlockSpec((BLK, FH), lambda ph, j: (j, 0)),
        ],
        out_shape=[
            jax.ShapeDtypeStruct((N, F), jnp.float32),
            jax.ShapeDtypeStruct((N, FH), jnp.float32),
            jax.ShapeDtypeStruct((N, FH), jnp.float32),
        ],
        scratch_shapes=[pltpu.VMEM((8, 128), jnp.float32),
                        pltpu.VMEM((8, 128), jnp.float32),
                        pltpu.VMEM((N, F), jnp.float32)],
    )


def _conn_body(adj_ref, cx_ref, w1r_ref, b1_ref, w1t_ref, a1_ref,
               w2r_ref, b2_ref, w2t_ref, a2_ref,
               w3r_ref, b3_ref, w3t_ref, a3_ref, out_ref):
    a = adj_ref[0]
    x = cx_ref[0]

    def prelu(z, al):
        return jnp.where(z >= 0, z, al * z)

    t = jnp.dot(a, x, preferred_element_type=jnp.float32)
    h = (jnp.dot(t, w1r_ref[...], preferred_element_type=jnp.float32)
         + b1_ref[...]
         + jnp.dot(x, w1t_ref[...], preferred_element_type=jnp.float32))
    h = prelu(h, a1_ref[...])
    t = jnp.dot(a, h, preferred_element_type=jnp.float32)
    h = (jnp.dot(t, w2r_ref[...], preferred_element_type=jnp.float32)
         + b2_ref[...]
         + jnp.dot(h, w2t_ref[...], preferred_element_type=jnp.float32))
    h = prelu(h, a2_ref[...])
    t = jnp.dot(a, h, preferred_element_type=jnp.float32)
    h = (jnp.dot(t, w3r_ref[...], preferred_element_type=jnp.float32)
         + b3_ref[...]
         + jnp.dot(h, w3t_ref[...], preferred_element_type=jnp.float32))
    h = prelu(h, a3_ref[...])
    out_ref[0] = h


def _make_conn():
    full = lambda r, c: pl.BlockSpec((r, c), lambda g: (0, 0))
    return pl.pallas_call(
        _conn_body,
        grid=(G,),
        in_specs=[
            pl.BlockSpec((1, 87, 87), lambda g: (g, 0, 0)),
            pl.BlockSpec((1, 87, 10), lambda g: (g, 0, 0)),
            full(10, 20), full(1, 20), full(10, 20), full(1, 1),
            full(20, 20), full(1, 20), full(20, 20), full(1, 1),
            full(20, 5), full(1, 5), full(20, 5), full(1, 1),
        ],
        out_specs=pl.BlockSpec((1, 87, 5), lambda g: (g, 0, 0)),
        out_shape=jax.ShapeDtypeStruct((G, 87, 5), jnp.float32),
    )


def _final_body(p_ref, h_ref, dis_ref, fb_ref, g_ref, b_ref, batch_ref,
                conn_ref, w1a_ref, w1b_ref, hb1_ref, hg_ref, hbb_ref,
                w2_ref, hb2_ref, out_ref, acc_ref, seg_ref, cnt_ref):
    j = pl.program_id(0)
    p = p_ref[...]
    h = h_ref[...]
    dis = dis_ref[...]
    u = dis * p + h * (dis * dis) + fb_ref[...]
    u = jnp.maximum(u, 0.0)

    @pl.when(j == 0)
    def _():
        acc_ref[...] = jnp.zeros_like(acc_ref)
        seg_ref[...] = jnp.zeros_like(seg_ref)
        cnt_ref[...] = jnp.zeros_like(cnt_ref)

    acc_ref[0:1, 0:F] += jnp.sum(u, axis=0, keepdims=True)
    acc_ref[1:2, 0:F] += jnp.sum(u * u, axis=0, keepdims=True)

    b = batch_ref[0]
    mask = (lax.broadcasted_iota(jnp.int32, (G, BLK), 0) == b
            ).astype(jnp.float32)
    seg_ref[...] += jnp.dot(mask, u, preferred_element_type=jnp.float32)
    cnt_ref[...] += jnp.sum(mask, axis=1, keepdims=True)

    @pl.when(j == NB - 1)
    def _():
        m = acc_ref[0:1, 0:F] / N
        v = acc_ref[1:2, 0:F] / N - m * m
        rstd = lax.rsqrt(v + 1e-5)
        cnt = jnp.maximum(cnt_ref[...], 1.0)
        mean_u = seg_ref[...] / cnt
        mesh_feat = (mean_u - m) * rstd * g_ref[...] + b_ref[...]
        z = (jnp.dot(mesh_feat, w1a_ref[...],
                     preferred_element_type=jnp.float32)
             + jnp.dot(conn_ref[...], w1b_ref[...],
                       preferred_element_type=jnp.float32)
             + hb1_ref[...])
        z = jnp.maximum(z, 0.0)
        zm = jnp.mean(z, axis=0, keepdims=True)
        zv = jnp.mean(z * z, axis=0, keepdims=True) - zm * zm
        zn = (z - zm) * lax.rsqrt(zv + 1e-5) * hg_ref[...] + hbb_ref[...]
        out_ref[...] = (jnp.dot(zn, w2_ref[...],
                                preferred_element_type=jnp.float32)
                        + hb2_ref[...])


def _make_final():
    full = lambda r, c: pl.BlockSpec((r, c), lambda j: (0, 0))
    return pl.pallas_call(
        _final_body,
        grid=(NB,),
        in_specs=[
            pl.BlockSpec((BLK, F), lambda j: (j, 0)),
            pl.BlockSpec((BLK, F), lambda j: (j, 0)),
            pl.BlockSpec((BLK, 1), lambda j: (j, 0)),
            full(1, F), full(1, F), full(1, F),
            pl.BlockSpec((1, 1, BLK), lambda j: (j, 0, 0)),
            full(G, 435), full(F, 10), full(435, 10), full(1, 10),
            full(1, 10), full(1, 10), full(10, 1), full(1, 1),
        ],
        out_specs=pl.BlockSpec((G, 1), lambda j: (0, 0)),
        out_shape=jax.ShapeDtypeStruct((G, 1), jnp.float32),
        scratch_shapes=[
            pltpu.VMEM((8, 128), jnp.float32),
            pltpu.VMEM((G, F), jnp.float32),
            pltpu.VMEM((G, 1), jnp.float32),
        ],
    )


_k1a = _make_k1a()
_k1 = _make_k1()
_klayer = _make_layer()
_kconn = _make_conn()
_kfinal = _make_final()


def kernel(mesh_pos, mesh_norm, mesh_dha, mesh_x, mesh_edge_index, mesh_batch,
           conn_x, conn_adj,
           cw1_rel, cb1, cw1_root, ca1,
           cw2_rel, cb2, cw2_root, ca2,
           cw3_rel, cb3, cw3_root, ca3,
           bn0_g, bn0_b,
           fw1, fb1, bn1_g, bn1_b,
           fw2, fb2, bn2_g, bn2_b,
           fw3, fb3, bn3_g, bn3_b,
           fw4, fb4, bn4_g, bn4_b,
           hw1, hb1, hbn_g, hbn_b, hw2, hb2):
    r = lambda a: a.reshape(1, -1)
    x13 = jnp.concatenate([mesh_pos, mesh_norm, mesh_dha, mesh_x], axis=1)
    zeros_h = jnp.zeros((NPAD, FH), jnp.float32)
    ones_h = jnp.ones((N, FH), jnp.float32)
    ones_c = jnp.ones((CH, FH), jnp.float32)
    mode0 = jnp.zeros((16,), jnp.int32)
    mode1 = jnp.ones((16,), jnp.int32)

    degp = _make_agg()(ones_h, ones_h, mesh_edge_index, zeros_h, ones_c,
                       mode0)

    conn3 = _kconn(conn_adj, conn_x,
                   cw1_rel, r(cb1), cw1_root, r(ca1),
                   cw2_rel, r(cb2), cw2_root, r(ca2),
                   cw3_rel, r(cb3), cw3_root, r(ca3))
    conn_feat = conn3.reshape(G, 435)

    sums0 = _k1a(x13)
    h, hs0, hs1, dis = _k1(x13, sums0, degp, r(bn0_g), r(bn0_b), fw1)

    fbs = (fb1, fb2, fb3)
    gs = (bn1_g, bn2_g, bn3_g)
    bs = (bn1_b, bn2_b, bn3_b)
    ws = (fw2, fw3, fw4)
    for i in range(3):
        p = _make_agg()(hs0, hs1, mesh_edge_index, zeros_h, ones_c, mode1)
        h, hs0, hs1 = _klayer(p, h, dis, r(fbs[i]), r(gs[i]), r(bs[i]), ws[i])

    p = _make_agg()(hs0, hs1, mesh_edge_index, zeros_h, ones_c, mode1)
    out = _kfinal(p, h, dis, r(fb4), r(bn4_g), r(bn4_b),
                  mesh_batch.reshape(NB, 1, BLK), conn_feat,
                  hw1[:F], hw1[F:], r(hb1), r(hbn_g), r(hbn_b), hw2, r(hb2))
    return out


# speedup vs baseline: 39.2768x; 1.0034x over previous
import functools

import jax
import jax.numpy as jnp
from jax import lax
from jax.experimental import pallas as pl
from jax.experimental.pallas import tpu as pltpu
from jax.experimental.pallas import tpu_sc as plsc

N = 50000
E = 1600000
G = 64
F = 32
NC, NS = 2, 16
NW = NC * NS
EPW = E // NW
CH = 1000
NPAD = 50048
ROWS_W = NPAD // NS
DEGW = 16

BLK = 5000
NB = N // BLK


FH = F // NC
EPS = E // NS
NCHS = EPS // CH


NBUF = 4


def _agg_body(hs_hbm, ei_hbm, zeros_hbm, ones_hbm, mode_hbm,
              out_hbm, sidx0, sidx1, sidx2, sidx3, didx0, didx1, didx2, didx3,
              rows0, rows1, rows2, rows3, agg_sh, mode_sm,
              sg0, sg1, sg2, sg3, ss0, ss1, ss2, ss3):
    c = lax.axis_index("c")
    s = lax.axis_index("s")
    sidx = (sidx0, sidx1, sidx2, sidx3)
    didx = (didx0, didx1, didx2, didx3)
    rows = (rows0, rows1, rows2, rows3)
    semg = (sg0, sg1, sg2, sg3)
    sems = (ss0, ss1, ss2, ss3)

    pltpu.sync_copy(mode_hbm, mode_sm)
    do_gather = jnp.max(mode_sm[...]) == 1

    def load_idx(i, b):
        base = s * EPS + i * CH

        @pl.when(do_gather)
        def _():
            pltpu.sync_copy(ei_hbm.at[0, pl.ds(base, CH)], sidx[b])

        pltpu.sync_copy(ei_hbm.at[1, pl.ds(base, CH)], didx[b])

    def start_gather(b):
        @pl.when(jnp.logical_and(do_gather, c == 0))
        def _():
            pltpu.async_copy(hs_hbm.at[0].at[sidx[b]], rows[b], semg[b])

        @pl.when(jnp.logical_and(do_gather, c == 1))
        def _():
            pltpu.async_copy(hs_hbm.at[1].at[sidx[b]], rows[b], semg[b])

    def wait_gather(b):
        @pl.when(do_gather)
        def _():
            pltpu.make_async_copy(hs_hbm.at[0].at[sidx[b]], rows[b],
                                  semg[b]).wait()

    def start_scatter(b):
        pltpu.async_copy(rows[b], agg_sh.at[didx[b]], sems[b], add=True)

    def wait_scatter(b):
        pltpu.make_async_copy(rows[b], agg_sh.at[didx[b]], sems[b]).wait()

    pltpu.sync_copy(zeros_hbm.at[pl.ds(s * ROWS_W, ROWS_W)],
                    agg_sh.at[pl.ds(s * ROWS_W, ROWS_W)])

    @pl.when(jnp.logical_not(do_gather))
    def _():
        for b in range(NBUF):
            pltpu.sync_copy(ones_hbm, rows[b])

    plsc.subcore_barrier()

    for j in range(2):
        load_idx(j, j)
        start_gather(j)

    def body(k, carry):
        for b in range(NBUF):
            i = k * NBUF + b
            j = i + 2
            bj = (b + 2) % NBUF

            @pl.when(j < NCHS)
            def _():
                @pl.when(j >= NBUF)
                def _():
                    wait_scatter(bj)

                load_idx(j, bj)
                start_gather(bj)

            wait_gather(b)
            start_scatter(b)
        return carry

    lax.fori_loop(0, NCHS // NBUF, body, 0)
    for b in range(NBUF):
        wait_scatter(b)
    plsc.subcore_barrier()
    pltpu.sync_copy(agg_sh.at[pl.ds(s * ROWS_W, ROWS_W)],
                    out_hbm.at[pl.ds(s * ROWS_W, ROWS_W), pl.ds(c * FH, FH)])


@functools.lru_cache(maxsize=None)
def _make_agg():
    return pl.kernel(
        _agg_body,
        out_type=jax.ShapeDtypeStruct((NPAD, F), jnp.float32),
        mesh=plsc.VectorSubcoreMesh(core_axis_name="c", subcore_axis_name="s"),
        scratch_types=(
            [pltpu.VMEM((CH,), jnp.int32)] * 8
            + [pltpu.VMEM((CH, FH), jnp.float32)] * 4
            + [pltpu.VMEM_SHARED((NPAD, FH), jnp.float32),
               pltpu.VMEM((16,), jnp.int32)]
            + [pltpu.SemaphoreType.DMA] * 8
        ),
        compiler_params=pltpu.CompilerParams(use_tc_tiling_on_sc=False,
                                             needs_layout_passes=False),
    )


def _k1a_body(x_ref, sums_ref, acc_ref):
    j = pl.program_id(0)
    x = x_ref[...]

    @pl.when(j == 0)
    def _():
        acc_ref[...] = jnp.zeros_like(acc_ref)

    acc_ref[0:1, 0:13] += jnp.sum(x, axis=0, keepdims=True)
    acc_ref[1:2, 0:13] += jnp.sum(x * x, axis=0, keepdims=True)

    @pl.when(j == NB - 1)
    def _():
        sums_ref[...] = acc_ref[...]


def _make_k1a():
    return pl.pallas_call(
        _k1a_body,
        grid=(NB,),
        in_specs=[pl.BlockSpec((BLK, 13), lambda j: (j, 0))],
        out_specs=pl.BlockSpec((8, 128), lambda j: (0, 0)),
        out_shape=jax.ShapeDtypeStruct((8, 128), jnp.float32),
        scratch_shapes=[pltpu.VMEM((8, 128), jnp.float32)],
    )


def _k1_body(x_ref, sums_ref, degp_ref, g0_ref, b0_ref, w_ref,
             h_ref, hs_ref, dis_ref, st_ref):
    j = pl.program_id(0)
    dp = degp_ref[...]
    dis = lax.rsqrt(1.0 + dp[:, 0:1])
    dis_ref[...] = dis

    @pl.when(j == 0)
    def _():
        m = sums_ref[0:1, 0:13] / N
        v = sums_ref[1:2, 0:13] / N - m * m
        scale = lax.rsqrt(v + 1e-5) * g0_ref[...]
        st_ref[0:1, 0:13] = scale
        st_ref[1:2, 0:13] = b0_ref[...] - m * scale

    xn = x_ref[...] * st_ref[0:1, 0:13] + st_ref[1:2, 0:13]
    h = jnp.dot(xn, w_ref[...], preferred_element_type=jnp.float32)
    h_ref[...] = h
    hsc = h * dis
    hs_ref[0] = hsc[:, :FH]
    hs_ref[1] = hsc[:, FH:]


def _make_k1():
    return pl.pallas_call(
        _k1_body,
        grid=(NB,),
        in_specs=[
            pl.BlockSpec((BLK, 13), lambda j: (j, 0)),
            pl.BlockSpec((8, 128), lambda j: (0, 0)),
            pl.BlockSpec((BLK, F), lambda j: (j, 0)),
            pl.BlockSpec((1, 13), lambda j: (0, 0)),
            pl.BlockSpec((1, 13), lambda j: (0, 0)),
            pl.BlockSpec((13, F), lambda j: (0, 0)),
        ],
        out_specs=[
            pl.BlockSpec((BLK, F), lambda j: (j, 0)),
            pl.BlockSpec((2, BLK, FH), lambda j: (0, j, 0)),
            pl.BlockSpec((BLK, 1), lambda j: (j, 0)),
        ],
        out_shape=[
            jax.ShapeDtypeStruct((N, F), jnp.float32),
            jax.ShapeDtypeStruct((2, N, FH), jnp.float32),
            jax.ShapeDtypeStruct((N, 1), jnp.float32),
        ],
        scratch_shapes=[pltpu.VMEM((8, 128), jnp.float32)],
    )


def _layer_body(p_ref, h_ref, dis_ref, fb_ref, g_ref, b_ref, w_ref,
                hn_ref, hs_ref, acc_ref, st_ref, u_ref):
    ph = pl.program_id(0)
    j = pl.program_id(1)
    dis = dis_ref[...]

    @pl.when(ph == 0)
    def _():
        u = dis * p_ref[...] + h_ref[...] * (dis * dis) + fb_ref[...]
        u = jnp.maximum(u, 0.0)
        u_ref[pl.ds(j * BLK, BLK), :] = u

        @pl.when(j == 0)
        def _():
            acc_ref[...] = jnp.zeros_like(acc_ref)

        acc_ref[0:1, 0:F] += jnp.sum(u, axis=0, keepdims=True)
        acc_ref[1:2, 0:F] += jnp.sum(u * u, axis=0, keepdims=True)

    @pl.when(ph == 1)
    def _():
        @pl.when(j == 0)
        def _():
            m = acc_ref[0:1, 0:F] / N
            v = acc_ref[1:2, 0:F] / N - m * m
            scale = lax.rsqrt(v + 1e-5) * g_ref[...]
            st_ref[0:1, 0:F] = scale
            st_ref[1:2, 0:F] = b_ref[...] - m * scale

        mx = u_ref[pl.ds(j * BLK, BLK), :] * st_ref[0:1, 0:F] + st_ref[1:2, 0:F]
        hn = jnp.dot(mx, w_ref[...], preferred_element_type=jnp.float32)
        hn_ref[...] = hn
        hsc = hn * dis
        hs_ref[0] = hsc[:, :FH]
        hs_ref[1] = hsc[:, FH:]


def _make_layer():
    return pl.pallas_call(
        _layer_body,
        grid=(2, NB),
        in_specs=[
            pl.BlockSpec((BLK, F), lambda ph, j: ((1 - ph) * j, 0)),
            pl.BlockSpec((BLK, F), lambda ph, j: ((1 - ph) * j, 0)),
            pl.BlockSpec((BLK, 1), lambda ph, j: (j, 0)),
            pl.BlockSpec((1, F), lambda ph, j: (0, 0)),
            pl.BlockSpec((1, F), lambda ph, j: (0, 0)),
            pl.BlockSpec((1, F), lambda ph, j: (0, 0)),
            pl.BlockSpec((F, F), lambda ph, j: (0, 0)),
        ],
        out_specs=[
            pl.BlockSpec((BLK, F), lambda ph, j: (j, 0)),
            pl.BlockSpec((2, BLK, FH), lambda ph, j: (0, j, 0)),
        ],
        out_shape=[
            jax.ShapeDtypeStruct((N, F), jnp.float32),
            jax.ShapeDtypeStruct((2, N, FH), jnp.float32),
        ],
        scratch_shapes=[pltpu.VMEM((8, 128), jnp.float32),
                        pltpu.VMEM((8, 128), jnp.float32),
                        pltpu.VMEM((N, F), jnp.float32)],
    )


def _conn_body(adj_ref, cx_ref, w1r_ref, b1_ref, w1t_ref, a1_ref,
               w2r_ref, b2_ref, w2t_ref, a2_ref,
               w3r_ref, b3_ref, w3t_ref, a3_ref, out_ref):
    a = adj_ref[0]
    x = cx_ref[0]

    def prelu(z, al):
        return jnp.where(z >= 0, z, al * z)

    t = jnp.dot(a, x, preferred_element_type=jnp.float32)
    h = (jnp.dot(t, w1r_ref[...], preferred_element_type=jnp.float32)
         + b1_ref[...]
         + jnp.dot(x, w1t_ref[...], preferred_element_type=jnp.float32))
    h = prelu(h, a1_ref[...])
    t = jnp.dot(a, h, preferred_element_type=jnp.float32)
    h = (jnp.dot(t, w2r_ref[...], preferred_element_type=jnp.float32)
         + b2_ref[...]
         + jnp.dot(h, w2t_ref[...], preferred_element_type=jnp.float32))
    h = prelu(h, a2_ref[...])
    t = jnp.dot(a, h, preferred_element_type=jnp.float32)
    h = (jnp.dot(t, w3r_ref[...], preferred_element_type=jnp.float32)
         + b3_ref[...]
         + jnp.dot(h, w3t_ref[...], preferred_element_type=jnp.float32))
    h = prelu(h, a3_ref[...])
    out_ref[0] = h


def _make_conn():
    full = lambda r, c: pl.BlockSpec((r, c), lambda g: (0, 0))
    return pl.pallas_call(
        _conn_body,
        grid=(G,),
        in_specs=[
            pl.BlockSpec((1, 87, 87), lambda g: (g, 0, 0)),
            pl.BlockSpec((1, 87, 10), lambda g: (g, 0, 0)),
            full(10, 20), full(1, 20), full(10, 20), full(1, 1),
            full(20, 20), full(1, 20), full(20, 20), full(1, 1),
            full(20, 5), full(1, 5), full(20, 5), full(1, 1),
        ],
        out_specs=pl.BlockSpec((1, 87, 5), lambda g: (g, 0, 0)),
        out_shape=jax.ShapeDtypeStruct((G, 87, 5), jnp.float32),
    )


def _final_body(p_ref, h_ref, dis_ref, fb_ref, g_ref, b_ref, batch_ref,
                conn_ref, w1a_ref, w1b_ref, hb1_ref, hg_ref, hbb_ref,
                w2_ref, hb2_ref, out_ref, acc_ref, seg_ref, cnt_ref):
    j = pl.program_id(0)
    p = p_ref[...]
    h = h_ref[...]
    dis = dis_ref[...]
    u = dis * p + h * (dis * dis) + fb_ref[...]
    u = jnp.maximum(u, 0.0)

    @pl.when(j == 0)
    def _():
        acc_ref[...] = jnp.zeros_like(acc_ref)
        seg_ref[...] = jnp.zeros_like(seg_ref)
        cnt_ref[...] = jnp.zeros_like(cnt_ref)

    acc_ref[0:1, 0:F] += jnp.sum(u, axis=0, keepdims=True)
    acc_ref[1:2, 0:F] += jnp.sum(u * u, axis=0, keepdims=True)

    b = batch_ref[0]
    mask = (lax.broadcasted_iota(jnp.int32, (G, BLK), 0) == b
            ).astype(jnp.float32)
    seg_ref[...] += jnp.dot(mask, u, preferred_element_type=jnp.float32)
    cnt_ref[...] += jnp.sum(mask, axis=1, keepdims=True)

    @pl.when(j == NB - 1)
    def _():
        m = acc_ref[0:1, 0:F] / N
        v = acc_ref[1:2, 0:F] / N - m * m
        rstd = lax.rsqrt(v + 1e-5)
        cnt = jnp.maximum(cnt_ref[...], 1.0)
        mean_u = seg_ref[...] / cnt
        mesh_feat = (mean_u - m) * rstd * g_ref[...] + b_ref[...]
        z = (jnp.dot(mesh_feat, w1a_ref[...],
                     preferred_element_type=jnp.float32)
             + jnp.dot(conn_ref[...], w1b_ref[...],
                       preferred_element_type=jnp.float32)
             + hb1_ref[...])
        z = jnp.maximum(z, 0.0)
        zm = jnp.mean(z, axis=0, keepdims=True)
        zv = jnp.mean(z * z, axis=0, keepdims=True) - zm * zm
        zn = (z - zm) * lax.rsqrt(zv + 1e-5) * hg_ref[...] + hbb_ref[...]
        out_ref[...] = (jnp.dot(zn, w2_ref[...],
                                preferred_element_type=jnp.float32)
                        + hb2_ref[...])


def _make_final():
    full = lambda r, c: pl.BlockSpec((r, c), lambda j: (0, 0))
    return pl.pallas_call(
        _final_body,
        grid=(NB,),
        in_specs=[
            pl.BlockSpec((BLK, F), lambda j: (j, 0)),
            pl.BlockSpec((BLK, F), lambda j: (j, 0)),
            pl.BlockSpec((BLK, 1), lambda j: (j, 0)),
            full(1, F), full(1, F), full(1, F),
            pl.BlockSpec((1, 1, BLK), lambda j: (j, 0, 0)),
            full(G, 435), full(F, 10), full(435, 10), full(1, 10),
            full(1, 10), full(1, 10), full(10, 1), full(1, 1),
        ],
        out_specs=pl.BlockSpec((G, 1), lambda j: (0, 0)),
        out_shape=jax.ShapeDtypeStruct((G, 1), jnp.float32),
        scratch_shapes=[
            pltpu.VMEM((8, 128), jnp.float32),
            pltpu.VMEM((G, F), jnp.float32),
            pltpu.VMEM((G, 1), jnp.float32),
        ],
    )


_k1a = _make_k1a()
_k1 = _make_k1()
_klayer = _make_layer()
_kconn = _make_conn()
_kfinal = _make_final()


def kernel(mesh_pos, mesh_norm, mesh_dha, mesh_x, mesh_edge_index, mesh_batch,
           conn_x, conn_adj,
           cw1_rel, cb1, cw1_root, ca1,
           cw2_rel, cb2, cw2_root, ca2,
           cw3_rel, cb3, cw3_root, ca3,
           bn0_g, bn0_b,
           fw1, fb1, bn1_g, bn1_b,
           fw2, fb2, bn2_g, bn2_b,
           fw3, fb3, bn3_g, bn3_b,
           fw4, fb4, bn4_g, bn4_b,
           hw1, hb1, hbn_g, hbn_b, hw2, hb2):
    r = lambda a: a.reshape(1, -1)
    x13 = jnp.concatenate([mesh_pos, mesh_norm, mesh_dha, mesh_x], axis=1)
    zeros_h = jnp.zeros((NPAD, FH), jnp.float32)
    zeros_h2 = jnp.zeros((2, N, FH), jnp.float32)
    ones_c = jnp.ones((CH, FH), jnp.float32)
    mode0 = jnp.zeros((16,), jnp.int32)
    mode1 = jnp.ones((16,), jnp.int32)

    degp = _make_agg()(zeros_h2, mesh_edge_index, zeros_h, ones_c, mode0)

    conn3 = _kconn(conn_adj, conn_x,
                   cw1_rel, r(cb1), cw1_root, r(ca1),
                   cw2_rel, r(cb2), cw2_root, r(ca2),
                   cw3_rel, r(cb3), cw3_root, r(ca3))
    conn_feat = conn3.reshape(G, 435)

    sums0 = _k1a(x13)
    h, hs, dis = _k1(x13, sums0, degp, r(bn0_g), r(bn0_b), fw1)

    fbs = (fb1, fb2, fb3)
    gs = (bn1_g, bn2_g, bn3_g)
    bs = (bn1_b, bn2_b, bn3_b)
    ws = (fw2, fw3, fw4)
    for i in range(3):
        p = _make_agg()(hs, mesh_edge_index, zeros_h, ones_c, mode1)
        h, hs = _klayer(p, h, dis, r(fbs[i]), r(gs[i]), r(bs[i]), ws[i])

    p = _make_agg()(hs, mesh_edge_index, zeros_h, ones_c, mode1)
    out = _kfinal(p, h, dis, r(fb4), r(bn4_g), r(bn4_b),
                  mesh_batch.reshape(NB, 1, BLK), conn_feat,
                  hw1[:F], hw1[F:], r(hb1), r(hbn_g), r(hbn_b), hw2, r(hb2))
    return out
